# baseline (speedup 1.0000x reference)
import functools

import numpy as np
import jax
import jax.numpy as jnp
from jax import lax
from jax.experimental import pallas as pl
from jax.experimental.pallas import tpu as pltpu
from jax.experimental.pallas import tpu_sc as plsc

N = 10000
NPAD = 10240
E = 320000
C = 128
B = 512
NC = 2
NS = 16
NW = NC * NS
ET = E // NW
KE = 80
KE2 = 80
ET2 = 10240
NCH = ET2 // KE
EPAD = ET2 - ET
DUMP = NPAD - NW
FCAP = ET2 + 16
A2R = 640
ROWS_T = NPAD // NW
KA = 80
RPT = NPAD // NS
RB = 512
GRID = NPAD // RB
HI = NPAD // C
EB = 3200
EGRID = E // EB

_MESH = plsc.VectorSubcoreMesh(
    core_axis_name="c", subcore_axis_name="s", num_cores=NC, num_subcores=NS)
_SC_PARAMS = pltpu.CompilerParams(needs_layout_passes=False)


def _enc_gather_body(emb_h, nid_h, sp_h, dp_h,
                     erows_h, fsrc_h, fdst_h, fcnt_h,
                     nidv, rows, sidx, didx, fsb, fdb, cbuf, sem, sem2):
    cid = lax.axis_index("c")
    sid = lax.axis_index("s")
    w = sid * NC + cid
    base = w * ROWS_T
    pltpu.sync_copy(nid_h.at[pl.ds(base, ROWS_T)], nidv)
    for c in range(ROWS_T // KA):
        pltpu.async_copy(
            emb_h.at[nidv.at[pl.ds(c * KA, KA)]],
            rows.at[pl.ds(c * KA, KA)], sem)
    ecp0 = pltpu.async_copy(sp_h.at[pl.ds(w * ET2, ET2)], sidx, sem2)
    ecp1 = pltpu.async_copy(dp_h.at[pl.ds(w * ET2, ET2)], didx, sem2)
    for c in range(ROWS_T // KA):
        pltpu.make_async_copy(
            emb_h.at[pl.ds(0, KA)], rows.at[pl.ds(c * KA, KA)], sem).wait()
    pltpu.sync_copy(rows, erows_h.at[pl.ds(base, ROWS_T)])
    ecp0.wait()
    ecp1.wait()

    def fp(k, carry):
        fsb[pl.ds(k * 16, 16)] = jnp.zeros((16,), jnp.int32)
        fdb[pl.ds(k * 16, 16)] = jnp.broadcast_to(B + w, (16,))
        return carry
    lax.fori_loop(0, FCAP // 16, fp, jnp.int32(0))

    def comp(k, cnt):
        sv = sidx[pl.ds(k * 16, 16)]
        dv = didx[pl.ds(k * 16, 16)]
        m = dv < B
        plsc.store_compressed(fsb.at[pl.ds(cnt, 16)], sv, mask=m)
        plsc.store_compressed(fdb.at[pl.ds(cnt, 16)], dv, mask=m)
        return cnt + plsc.all_reduce_population_count(m)[0]

    cnt = lax.fori_loop(0, ET2 // 16, comp, jnp.int32(0))
    cbuf[...] = jnp.broadcast_to(cnt, (16,))
    pltpu.sync_copy(cbuf, fcnt_h.at[w])
    pltpu.sync_copy(fsb, fsrc_h.at[pl.ds(w * FCAP, FCAP)])
    pltpu.sync_copy(fdb, fdst_h.at[pl.ds(w * FCAP, FCAP)])


@jax.jit
def _enc_gather(emb, nid_p, sp, dp):
    return pl.kernel(
        _enc_gather_body,
        out_type=[
            jax.ShapeDtypeStruct((NPAD, C), jnp.float32),
            jax.ShapeDtypeStruct((NW * FCAP,), jnp.int32),
            jax.ShapeDtypeStruct((NW * FCAP,), jnp.int32),
            jax.ShapeDtypeStruct((NW, 16), jnp.int32),
        ],
        mesh=_MESH,
        scratch_types=[
            pltpu.VMEM((ROWS_T,), jnp.int32),
            pltpu.VMEM((ROWS_T, C), jnp.float32),
            pltpu.VMEM((ET2,), jnp.int32),
            pltpu.VMEM((ET2,), jnp.int32),
            pltpu.VMEM((FCAP,), jnp.int32),
            pltpu.VMEM((FCAP,), jnp.int32),
            pltpu.VMEM((16,), jnp.int32),
            pltpu.SemaphoreType.DMA,
            pltpu.SemaphoreType.DMA,
        ],
        compiler_params=_SC_PARAMS,
    )(emb, nid_p, sp, dp)


def _zero_rows(buf, nrows):
    def zr(k, carry):
        buf[k // (C // 16), pl.ds((k % (C // 16)) * 16, 16)] = (
            jnp.zeros((16,), jnp.float32))
        return carry
    lax.fori_loop(0, nrows * (C // 16), zr, jnp.int32(0))


def _msg1_body(hp_h, sp_h, dp_h, zrows_h, part_h, acc, sbuf, dbuf, rows0,
               sem0):
    cid = lax.axis_index("c")
    sid = lax.axis_index("s")
    w = sid * NC + cid
    base = w * ET2
    pltpu.sync_copy(zrows_h, acc.at[pl.ds(sid * RPT, RPT)])
    plsc.subcore_barrier()

    def body(c, carry):
        off = base + c * KE
        pltpu.sync_copy(sp_h.at[pl.ds(off, KE)], sbuf)
        pltpu.sync_copy(dp_h.at[pl.ds(off, KE)], dbuf)
        pltpu.async_copy(hp_h.at[sbuf], rows0, sem0).wait()
        pltpu.sync_copy(rows0, acc.at[dbuf], add=True)
        return carry

    lax.fori_loop(0, NCH, body, jnp.int32(0))
    plsc.subcore_barrier()
    pltpu.sync_copy(acc.at[pl.ds(sid * RPT, RPT)],
                    part_h.at[cid, pl.ds(sid * RPT, RPT)])


@jax.jit
def _msg1(hp, sp, dp, zrows):
    return pl.kernel(
        _msg1_body,
        out_type=jax.ShapeDtypeStruct((NC, NPAD, C), jnp.float32),
        mesh=_MESH,
        scratch_types=[
            pltpu.VMEM_SHARED((NPAD, C), jnp.float32),
            pltpu.VMEM((KE,), jnp.int32),
            pltpu.VMEM((KE,), jnp.int32),
            pltpu.VMEM((KE, C), jnp.float32),
            pltpu.SemaphoreType.DMA,
        ],
    )(hp, sp, dp, zrows)


def _msg2_body(h1_h, fsrc_h, fdst_h, fcnt_h, z2_h, part2_h,
               acc2, sbuf, dbuf, rows, cntv, sem):
    cid = lax.axis_index("c")
    sid = lax.axis_index("s")
    w = sid * NC + cid
    pltpu.sync_copy(z2_h, acc2.at[pl.ds(sid * (A2R // NS), A2R // NS)])
    pltpu.sync_copy(fcnt_h.at[w], cntv)
    plsc.subcore_barrier()
    cnt = cntv[...][0]
    nch = (cnt + (KE2 - 1)) // KE2

    def body(c, carry):
        pltpu.sync_copy(fsrc_h.at[pl.ds(w * FCAP + c * KE2, KE2)], sbuf)
        pltpu.sync_copy(fdst_h.at[pl.ds(w * FCAP + c * KE2, KE2)], dbuf)
        pltpu.async_copy(h1_h.at[sbuf], rows, sem).wait()
        pltpu.sync_copy(rows, acc2.at[dbuf], add=True)
        return carry

    lax.fori_loop(0, nch, body, jnp.int32(0))
    plsc.subcore_barrier()
    pltpu.sync_copy(acc2.at[pl.ds(sid * (A2R // NS), A2R // NS)],
                    part2_h.at[cid, pl.ds(sid * (A2R // NS), A2R // NS)])


@jax.jit
def _msg2(h1, fsrc, fdst, fcnt, z2):
    return pl.kernel(
        _msg2_body,
        out_type=jax.ShapeDtypeStruct((NC, A2R, C), jnp.float32),
        mesh=_MESH,
        scratch_types=[
            pltpu.VMEM_SHARED((A2R, C), jnp.float32),
            pltpu.VMEM((KE2,), jnp.int32),
            pltpu.VMEM((KE2,), jnp.int32),
            pltpu.VMEM((KE2, C), jnp.float32),
            pltpu.VMEM((16,), jnp.int32),
            pltpu.SemaphoreType.DMA,
        ],
        compiler_params=_SC_PARAMS,
    )(h1, fsrc, fdst, fcnt, z2)


def _deg_body(d_ref, out_ref):
    i = pl.program_id(0)

    @pl.when(i == 0)
    def _():
        out_ref[...] = jnp.zeros_like(out_ref)

    d = d_ref[...]
    hi = d // C
    lo = d % C
    oh_hi = (hi == lax.broadcasted_iota(jnp.int32, (EB, HI), 1)
             ).astype(jnp.float32)
    oh_lo = (lo == lax.broadcasted_iota(jnp.int32, (EB, C), 1)
             ).astype(jnp.float32)
    out_ref[...] += lax.dot_general(
        oh_hi, oh_lo, (((0,), (0,)), ((), ())),
        preferred_element_type=jnp.float32)


@jax.jit
def _deg_histogram(dst2):
    return pl.pallas_call(
        _deg_body,
        grid=(EGRID,),
        in_specs=[pl.BlockSpec((EB, 1), lambda i: (i, 0))],
        out_specs=pl.BlockSpec((HI, C), lambda i: (0, 0)),
        out_shape=jax.ShapeDtypeStruct((HI, C), jnp.float32),
    )(dst2)


def _h0_body(x_ref, bv_ref, nt_ref, seed_ref, er_ref, wenc_ref, wts_ref,
             wtc_ref, benc_ref, out_ref):
    onehot = (bv_ref[...] == lax.broadcasted_iota(jnp.int32, (RB, B), 1)
              ).astype(jnp.float32)
    st = jnp.dot(onehot, seed_ref[...], preferred_element_type=jnp.float32)
    rel = st - nt_ref[...]
    k = lax.broadcasted_iota(jnp.int32, (1, C // 2), 1).astype(jnp.float32)
    freqs = jnp.exp(k * (-np.log(10000.0) / (C // 2)))
    ang = rel * freqs
    out_ref[...] = (
        jnp.dot(x_ref[...], wenc_ref[...], preferred_element_type=jnp.float32)
        + jnp.dot(jnp.sin(ang), wts_ref[...],
                  preferred_element_type=jnp.float32)
        + jnp.dot(jnp.cos(ang), wtc_ref[...],
                  preferred_element_type=jnp.float32)
        + benc_ref[...] + er_ref[...])


@jax.jit
def _h0(x_p, bv2, nt2, seed2, erows, W_enc, Wt_sin, Wt_cos, b_enc2):
    return pl.pallas_call(
        _h0_body,
        grid=(GRID,),
        in_specs=[
            pl.BlockSpec((RB, C), lambda i: (i, 0)),
            pl.BlockSpec((RB, 1), lambda i: (i, 0)),
            pl.BlockSpec((RB, 1), lambda i: (i, 0)),
            pl.BlockSpec((B, 1), lambda i: (0, 0)),
            pl.BlockSpec((RB, C), lambda i: (i, 0)),
            pl.BlockSpec((C, C), lambda i: (0, 0)),
            pl.BlockSpec((C // 2, C), lambda i: (0, 0)),
            pl.BlockSpec((C // 2, C), lambda i: (0, 0)),
            pl.BlockSpec((1, C), lambda i: (0, 0)),
        ],
        out_specs=pl.BlockSpec((RB, C), lambda i: (i, 0)),
        out_shape=jax.ShapeDtypeStruct((NPAD, C), jnp.float32),
    )(x_p, bv2, nt2, seed2, erows, W_enc, Wt_sin, Wt_cos, b_enc2)


def _layer_body(h_ref, p0_ref, p1_ref, deg_ref, ws_ref, wn_ref, bg_ref,
                out_ref):
    p = p0_ref[0] + p1_ref[0]
    deg = jnp.maximum(deg_ref[...], 1.0)
    mean = p / deg
    out_ref[...] = jnp.maximum(
        jnp.dot(h_ref[...], ws_ref[...], preferred_element_type=jnp.float32)
        + jnp.dot(mean, wn_ref[...], preferred_element_type=jnp.float32)
        + bg_ref[...], 0.0)


@jax.jit
def _layer(h, part, deg2, ws, wn, bg2):
    return pl.pallas_call(
        _layer_body,
        grid=(GRID,),
        in_specs=[
            pl.BlockSpec((RB, C), lambda i: (i, 0)),
            pl.BlockSpec((1, RB, C), lambda i: (0, i, 0)),
            pl.BlockSpec((1, RB, C), lambda i: (1, i, 0)),
            pl.BlockSpec((RB, 1), lambda i: (i, 0)),
            pl.BlockSpec((C, C), lambda i: (0, 0)),
            pl.BlockSpec((C, C), lambda i: (0, 0)),
            pl.BlockSpec((1, C), lambda i: (0, 0)),
        ],
        out_specs=pl.BlockSpec((RB, C), lambda i: (i, 0)),
        out_shape=jax.ShapeDtypeStruct((NPAD, C), jnp.float32),
    )(h, part, part, deg2, ws, wn, bg2)


def _head_body(h_ref, p0_ref, p1_ref, deg_ref, ws_ref, wn_ref, bg_ref, wh_ref,
               bh_ref, out_ref):
    p = p0_ref[0] + p1_ref[0]
    deg = jnp.maximum(deg_ref[...], 1.0)
    mean = p / deg
    h2 = jnp.maximum(
        jnp.dot(h_ref[...], ws_ref[...], preferred_element_type=jnp.float32)
        + jnp.dot(mean, wn_ref[...], preferred_element_type=jnp.float32)
        + bg_ref[...], 0.0)
    out_ref[...] = (jnp.dot(h2, wh_ref[...], preferred_element_type=jnp.float32)
                    + bh_ref[...])


@jax.jit
def _head(h, part2, deg2, ws, wn, bg2, W_head, b_head2):
    return pl.pallas_call(
        _head_body,
        grid=(1,),
        in_specs=[
            pl.BlockSpec((B, C), lambda i: (0, 0)),
            pl.BlockSpec((1, B, C), lambda i: (0, 0, 0)),
            pl.BlockSpec((1, B, C), lambda i: (1, 0, 0)),
            pl.BlockSpec((B, 1), lambda i: (0, 0)),
            pl.BlockSpec((C, C), lambda i: (0, 0)),
            pl.BlockSpec((C, C), lambda i: (0, 0)),
            pl.BlockSpec((1, C), lambda i: (0, 0)),
            pl.BlockSpec((C, 1), lambda i: (0, 0)),
            pl.BlockSpec((1, 1), lambda i: (0, 0)),
        ],
        out_specs=pl.BlockSpec((B, 1), lambda i: (0, 0)),
        out_shape=jax.ShapeDtypeStruct((B, 1), jnp.float32),
    )(h, part2, part2, deg2, ws, wn, bg2, W_head, b_head2)


def kernel(x, edge_index, n_id, node_time, seed_time, batch_vec,
           W_enc, b_enc, W_time, emb, W_self, W_neigh, b_gnn, W_head, b_head):
    pad = NPAD - N
    nid_p = jnp.concatenate([n_id, jnp.zeros((pad,), jnp.int32)])
    bv_p = jnp.concatenate([batch_vec, jnp.zeros((pad,), jnp.int32)])
    nt_p = jnp.concatenate([node_time, jnp.zeros((pad,), jnp.float32)])
    x_p = jnp.concatenate([x, jnp.zeros((pad, C), jnp.float32)])
    src = edge_index[0]
    dst = edge_index[1]
    sp = jnp.concatenate(
        [src.reshape(NW, ET),
         jnp.zeros((NW, EPAD), jnp.int32)], axis=1).reshape(NW * ET2)
    dumps = DUMP + jnp.arange(NW, dtype=jnp.int32)[:, None]
    dp = jnp.concatenate(
        [dst.reshape(NW, ET),
         jnp.broadcast_to(dumps, (NW, EPAD))], axis=1).reshape(NW * ET2)

    deg2 = _deg_histogram(dst.reshape(E, 1)).reshape(NPAD, 1)
    erows, fsrc, fdst, fcnt = _enc_gather(emb, nid_p, sp, dp)
    h0 = _h0(x_p, bv_p.reshape(NPAD, 1), nt_p.reshape(NPAD, 1),
             seed_time.reshape(B, 1), erows,
             W_enc, W_time[:C // 2], W_time[C // 2:], b_enc.reshape(1, C))
    part1 = _msg1(h0, sp, dp, jnp.zeros((RPT, C), jnp.float32))
    h1 = _layer(h0, part1, deg2, W_self[0], W_neigh[0], b_gnn[0].reshape(1, C))
    part2 = _msg2(h1, fsrc, fdst, fcnt,
                  jnp.zeros((A2R // NS, C), jnp.float32))
    return _head(h1, part2, deg2, W_self[1], W_neigh[1],
                 b_gnn[1].reshape(1, C), W_head, b_head.reshape(1, 1))


# speedup vs baseline: 1.5160x; 1.5160x over previous
import functools

import numpy as np
import jax
import jax.numpy as jnp
from jax import lax
from jax.experimental import pallas as pl
from jax.experimental.pallas import tpu as pltpu
from jax.experimental.pallas import tpu_sc as plsc

N = 10000
NPAD = 10240
E = 320000
C = 128
B = 512
NC = 2
NS = 16
NW = NC * NS
ET = E // NW
KE = 80
KE2 = 80
NCH = ET // KE
FCAP = ET + 16
A2R = 640
ROWS_T = NPAD // NW
KA = 80
RPT = NPAD // NS
RB = 512
GRID = NPAD // RB
HI = NPAD // C
EB = 3200
EGRID = E // EB

_MESH = plsc.VectorSubcoreMesh(
    core_axis_name="c", subcore_axis_name="s", num_cores=NC, num_subcores=NS)
_SC_PARAMS = pltpu.CompilerParams(needs_layout_passes=False)


def _enc_gather_body(emb_h, nid_h, sp_h, dp_h,
                     erows_h, fsrc_h, fdst_h, fcnt_h,
                     nidv, rows, sidx, didx, fsb, fdb, cbuf, sem, sem2):
    cid = lax.axis_index("c")
    sid = lax.axis_index("s")
    w = sid * NC + cid
    base = w * ROWS_T
    pltpu.sync_copy(nid_h.at[pl.ds(base, ROWS_T)], nidv)
    for c in range(ROWS_T // KA):
        pltpu.async_copy(
            emb_h.at[nidv.at[pl.ds(c * KA, KA)]],
            rows.at[pl.ds(c * KA, KA)], sem)
    ecp0 = pltpu.async_copy(sp_h.at[pl.ds(w * ET, ET)], sidx, sem2)
    ecp1 = pltpu.async_copy(dp_h.at[pl.ds(w * ET, ET)], didx, sem2)
    for c in range(ROWS_T // KA):
        pltpu.make_async_copy(
            emb_h.at[pl.ds(0, KA)], rows.at[pl.ds(c * KA, KA)], sem).wait()
    pltpu.sync_copy(rows, erows_h.at[pl.ds(base, ROWS_T)])
    ecp0.wait()
    ecp1.wait()

    def fp(k, carry):
        fsb[pl.ds(k * 16, 16)] = jnp.zeros((16,), jnp.int32)
        fdb[pl.ds(k * 16, 16)] = jnp.broadcast_to(B + w, (16,))
        return carry
    lax.fori_loop(0, FCAP // 16, fp, jnp.int32(0))

    def comp(k, cnt):
        sv = sidx[pl.ds(k * 16, 16)]
        dv = didx[pl.ds(k * 16, 16)]
        m = dv < B
        plsc.store_compressed(fsb.at[pl.ds(cnt, 16)], sv, mask=m)
        plsc.store_compressed(fdb.at[pl.ds(cnt, 16)], dv, mask=m)
        return cnt + plsc.all_reduce_population_count(m)[0]

    cnt = lax.fori_loop(0, ET // 16, comp, jnp.int32(0))
    cbuf[...] = jnp.broadcast_to(cnt, (16,))
    pltpu.sync_copy(cbuf, fcnt_h.at[w])
    pltpu.sync_copy(fsb, fsrc_h.at[pl.ds(w * FCAP, FCAP)])
    pltpu.sync_copy(fdb, fdst_h.at[pl.ds(w * FCAP, FCAP)])


@jax.jit
def _enc_gather(emb, nid_p, sp, dp):
    return pl.kernel(
        _enc_gather_body,
        out_type=[
            jax.ShapeDtypeStruct((NPAD, C), jnp.float32),
            jax.ShapeDtypeStruct((NW * FCAP,), jnp.int32),
            jax.ShapeDtypeStruct((NW * FCAP,), jnp.int32),
            jax.ShapeDtypeStruct((NW, 16), jnp.int32),
        ],
        mesh=_MESH,
        scratch_types=[
            pltpu.VMEM((ROWS_T,), jnp.int32),
            pltpu.VMEM((ROWS_T, C), jnp.float32),
            pltpu.VMEM((ET,), jnp.int32),
            pltpu.VMEM((ET,), jnp.int32),
            pltpu.VMEM((FCAP,), jnp.int32),
            pltpu.VMEM((FCAP,), jnp.int32),
            pltpu.VMEM((16,), jnp.int32),
            pltpu.SemaphoreType.DMA,
            pltpu.SemaphoreType.DMA,
        ],
        compiler_params=_SC_PARAMS,
    )(emb, nid_p, sp, dp)


def _zero_rows(buf, nrows):
    def zr(k, carry):
        buf[k // (C // 16), pl.ds((k % (C // 16)) * 16, 16)] = (
            jnp.zeros((16,), jnp.float32))
        return carry
    lax.fori_loop(0, nrows * (C // 16), zr, jnp.int32(0))


def _msg1_body(hp_h, sp_h, dp_h, zrows_h, part_h, acc, sbuf, dbuf, rows0,
               sem0):
    cid = lax.axis_index("c")
    sid = lax.axis_index("s")
    w = sid * NC + cid
    base = w * ET
    pltpu.sync_copy(zrows_h, acc.at[pl.ds(sid * RPT, RPT)])
    plsc.subcore_barrier()

    def body(c, carry):
        off = base + c * KE
        pltpu.sync_copy(sp_h.at[pl.ds(off, KE)], sbuf)
        pltpu.sync_copy(dp_h.at[pl.ds(off, KE)], dbuf)
        pltpu.async_copy(hp_h.at[sbuf], rows0, sem0).wait()
        pltpu.sync_copy(rows0, acc.at[dbuf], add=True)
        return carry

    lax.fori_loop(0, NCH, body, jnp.int32(0))
    plsc.subcore_barrier()
    pltpu.sync_copy(acc.at[pl.ds(sid * RPT, RPT)],
                    part_h.at[cid, pl.ds(sid * RPT, RPT)])


@jax.jit
def _msg1(hp, sp, dp, zrows):
    return pl.kernel(
        _msg1_body,
        out_type=jax.ShapeDtypeStruct((NC, NPAD, C), jnp.float32),
        mesh=_MESH,
        scratch_types=[
            pltpu.VMEM_SHARED((NPAD, C), jnp.float32),
            pltpu.VMEM((KE,), jnp.int32),
            pltpu.VMEM((KE,), jnp.int32),
            pltpu.VMEM((KE, C), jnp.float32),
            pltpu.SemaphoreType.DMA,
        ],
    )(hp, sp, dp, zrows)


def _msg2_body(h1_h, fsrc_h, fdst_h, fcnt_h, z2_h, part2_h,
               acc2, sbuf, dbuf, rows, cntv, sem):
    cid = lax.axis_index("c")
    sid = lax.axis_index("s")
    w = sid * NC + cid
    pltpu.sync_copy(z2_h, acc2.at[pl.ds(sid * (A2R // NS), A2R // NS)])
    pltpu.sync_copy(fcnt_h.at[w], cntv)
    plsc.subcore_barrier()
    cnt = cntv[...][0]
    nch = (cnt + (KE2 - 1)) // KE2

    def body(c, carry):
        pltpu.sync_copy(fsrc_h.at[pl.ds(w * FCAP + c * KE2, KE2)], sbuf)
        pltpu.sync_copy(fdst_h.at[pl.ds(w * FCAP + c * KE2, KE2)], dbuf)
        pltpu.async_copy(h1_h.at[sbuf], rows, sem).wait()
        pltpu.sync_copy(rows, acc2.at[dbuf], add=True)
        return carry

    lax.fori_loop(0, nch, body, jnp.int32(0))
    plsc.subcore_barrier()
    pltpu.sync_copy(acc2.at[pl.ds(sid * (A2R // NS), A2R // NS)],
                    part2_h.at[cid, pl.ds(sid * (A2R // NS), A2R // NS)])


@jax.jit
def _msg2(h1, fsrc, fdst, fcnt, z2):
    return pl.kernel(
        _msg2_body,
        out_type=jax.ShapeDtypeStruct((NC, A2R, C), jnp.float32),
        mesh=_MESH,
        scratch_types=[
            pltpu.VMEM_SHARED((A2R, C), jnp.float32),
            pltpu.VMEM((KE2,), jnp.int32),
            pltpu.VMEM((KE2,), jnp.int32),
            pltpu.VMEM((KE2, C), jnp.float32),
            pltpu.VMEM((16,), jnp.int32),
            pltpu.SemaphoreType.DMA,
        ],
        compiler_params=_SC_PARAMS,
    )(h1, fsrc, fdst, fcnt, z2)


def _deg_body(d_ref, out_ref):
    i = pl.program_id(0)

    @pl.when(i == 0)
    def _():
        out_ref[...] = jnp.zeros_like(out_ref)

    d = d_ref[...]
    hi = d // C
    lo = d % C
    oh_hi = (hi == lax.broadcasted_iota(jnp.int32, (EB, HI), 1)
             ).astype(jnp.float32)
    oh_lo = (lo == lax.broadcasted_iota(jnp.int32, (EB, C), 1)
             ).astype(jnp.float32)
    out_ref[...] += lax.dot_general(
        oh_hi, oh_lo, (((0,), (0,)), ((), ())),
        preferred_element_type=jnp.float32)


@jax.jit
def _deg_histogram(dst2):
    return pl.pallas_call(
        _deg_body,
        grid=(EGRID,),
        in_specs=[pl.BlockSpec((EB, 1), lambda i: (i, 0))],
        out_specs=pl.BlockSpec((HI, C), lambda i: (0, 0)),
        out_shape=jax.ShapeDtypeStruct((HI, C), jnp.float32),
    )(dst2)


def _h0_body(x_ref, bv_ref, nt_ref, seed_ref, er_ref, wenc_ref, wts_ref,
             wtc_ref, benc_ref, out_ref):
    onehot = (bv_ref[...] == lax.broadcasted_iota(jnp.int32, (RB, B), 1)
              ).astype(jnp.float32)
    st = jnp.dot(onehot, seed_ref[...], preferred_element_type=jnp.float32)
    rel = st - nt_ref[...]
    k = lax.broadcasted_iota(jnp.int32, (1, C // 2), 1).astype(jnp.float32)
    freqs = jnp.exp(k * (-np.log(10000.0) / (C // 2)))
    ang = rel * freqs
    out_ref[...] = (
        jnp.dot(x_ref[...], wenc_ref[...], preferred_element_type=jnp.float32)
        + jnp.dot(jnp.sin(ang), wts_ref[...],
                  preferred_element_type=jnp.float32)
        + jnp.dot(jnp.cos(ang), wtc_ref[...],
                  preferred_element_type=jnp.float32)
        + benc_ref[...] + er_ref[...])


@jax.jit
def _h0(x_p, bv2, nt2, seed2, erows, W_enc, Wt_sin, Wt_cos, b_enc2):
    return pl.pallas_call(
        _h0_body,
        grid=(GRID,),
        in_specs=[
            pl.BlockSpec((RB, C), lambda i: (i, 0)),
            pl.BlockSpec((RB, 1), lambda i: (i, 0)),
            pl.BlockSpec((RB, 1), lambda i: (i, 0)),
            pl.BlockSpec((B, 1), lambda i: (0, 0)),
            pl.BlockSpec((RB, C), lambda i: (i, 0)),
            pl.BlockSpec((C, C), lambda i: (0, 0)),
            pl.BlockSpec((C // 2, C), lambda i: (0, 0)),
            pl.BlockSpec((C // 2, C), lambda i: (0, 0)),
            pl.BlockSpec((1, C), lambda i: (0, 0)),
        ],
        out_specs=pl.BlockSpec((RB, C), lambda i: (i, 0)),
        out_shape=jax.ShapeDtypeStruct((NPAD, C), jnp.float32),
    )(x_p, bv2, nt2, seed2, erows, W_enc, Wt_sin, Wt_cos, b_enc2)


def _layer_body(h_ref, p0_ref, p1_ref, deg_ref, ws_ref, wn_ref, bg_ref,
                out_ref):
    p = p0_ref[0] + p1_ref[0]
    deg = jnp.maximum(deg_ref[...], 1.0)
    mean = p / deg
    out_ref[...] = jnp.maximum(
        jnp.dot(h_ref[...], ws_ref[...], preferred_element_type=jnp.float32)
        + jnp.dot(mean, wn_ref[...], preferred_element_type=jnp.float32)
        + bg_ref[...], 0.0)


@jax.jit
def _layer(h, part, deg2, ws, wn, bg2):
    return pl.pallas_call(
        _layer_body,
        grid=(GRID,),
        in_specs=[
            pl.BlockSpec((RB, C), lambda i: (i, 0)),
            pl.BlockSpec((1, RB, C), lambda i: (0, i, 0)),
            pl.BlockSpec((1, RB, C), lambda i: (1, i, 0)),
            pl.BlockSpec((RB, 1), lambda i: (i, 0)),
            pl.BlockSpec((C, C), lambda i: (0, 0)),
            pl.BlockSpec((C, C), lambda i: (0, 0)),
            pl.BlockSpec((1, C), lambda i: (0, 0)),
        ],
        out_specs=pl.BlockSpec((RB, C), lambda i: (i, 0)),
        out_shape=jax.ShapeDtypeStruct((NPAD, C), jnp.float32),
    )(h, part, part, deg2, ws, wn, bg2)


def _head_body(h_ref, p0_ref, p1_ref, deg_ref, ws_ref, wn_ref, bg_ref, wh_ref,
               bh_ref, out_ref):
    p = p0_ref[0] + p1_ref[0]
    deg = jnp.maximum(deg_ref[...], 1.0)
    mean = p / deg
    h2 = jnp.maximum(
        jnp.dot(h_ref[...], ws_ref[...], preferred_element_type=jnp.float32)
        + jnp.dot(mean, wn_ref[...], preferred_element_type=jnp.float32)
        + bg_ref[...], 0.0)
    out_ref[...] = (jnp.dot(h2, wh_ref[...], preferred_element_type=jnp.float32)
                    + bh_ref[...])


@jax.jit
def _head(h, part2, deg2, ws, wn, bg2, W_head, b_head2):
    return pl.pallas_call(
        _head_body,
        grid=(1,),
        in_specs=[
            pl.BlockSpec((B, C), lambda i: (0, 0)),
            pl.BlockSpec((1, B, C), lambda i: (0, 0, 0)),
            pl.BlockSpec((1, B, C), lambda i: (1, 0, 0)),
            pl.BlockSpec((B, 1), lambda i: (0, 0)),
            pl.BlockSpec((C, C), lambda i: (0, 0)),
            pl.BlockSpec((C, C), lambda i: (0, 0)),
            pl.BlockSpec((1, C), lambda i: (0, 0)),
            pl.BlockSpec((C, 1), lambda i: (0, 0)),
            pl.BlockSpec((1, 1), lambda i: (0, 0)),
        ],
        out_specs=pl.BlockSpec((B, 1), lambda i: (0, 0)),
        out_shape=jax.ShapeDtypeStruct((B, 1), jnp.float32),
    )(h, part2, part2, deg2, ws, wn, bg2, W_head, b_head2)


def kernel(x, edge_index, n_id, node_time, seed_time, batch_vec,
           W_enc, b_enc, W_time, emb, W_self, W_neigh, b_gnn, W_head, b_head):
    pad = NPAD - N
    nid_p = jnp.concatenate([n_id, jnp.zeros((pad,), jnp.int32)])
    bv_p = jnp.concatenate([batch_vec, jnp.zeros((pad,), jnp.int32)])
    nt_p = jnp.concatenate([node_time, jnp.zeros((pad,), jnp.float32)])
    x_p = jnp.concatenate([x, jnp.zeros((pad, C), jnp.float32)])
    src = edge_index[0]
    dst = edge_index[1]
    sp = src
    dp = dst

    deg2 = _deg_histogram(dst.reshape(E, 1)).reshape(NPAD, 1)
    erows, fsrc, fdst, fcnt = _enc_gather(emb, nid_p, sp, dp)
    h0 = _h0(x_p, bv_p.reshape(NPAD, 1), nt_p.reshape(NPAD, 1),
             seed_time.reshape(B, 1), erows,
             W_enc, W_time[:C // 2], W_time[C // 2:], b_enc.reshape(1, C))
    part1 = _msg1(h0, sp, dp, jnp.zeros((RPT, C), jnp.float32))
    h1 = _layer(h0, part1, deg2, W_self[0], W_neigh[0], b_gnn[0].reshape(1, C))
    part2 = _msg2(h1, fsrc, fdst, fcnt,
                  jnp.zeros((A2R // NS, C), jnp.float32))
    return _head(h1, part2, deg2, W_self[1], W_neigh[1],
                 b_gnn[1].reshape(1, C), W_head, b_head.reshape(1, 1))


# speedup vs baseline: 1.7472x; 1.1525x over previous
import functools

import numpy as np
import jax
import jax.numpy as jnp
from jax import lax
from jax.experimental import pallas as pl
from jax.experimental.pallas import tpu as pltpu
from jax.experimental.pallas import tpu_sc as plsc

N = 10000
NPAD = 10240
E = 320000
C = 128
B = 512
NC = 2
NS = 16
NW = NC * NS
ET = E // NW
KE = 80
KE2 = 80
NCH = ET // KE
FCAP = ET + 16
A2R = 640
ROWS_T = NPAD // NW
KA = 80
RPT = NPAD // NS
RB = 512
GRID = NPAD // RB
HI = NPAD // C
EB = 3200
EGRID = E // EB

_MESH = plsc.VectorSubcoreMesh(
    core_axis_name="c", subcore_axis_name="s", num_cores=NC, num_subcores=NS)
_SC_PARAMS = pltpu.CompilerParams(needs_layout_passes=False)


def _enc_gather_body(emb_h, nid_h, sp_h, dp_h,
                     erows_h, fsrc_h, fdst_h, fcnt_h,
                     nidv, rows, sidx, didx, fsb, fdb, cbuf, sem, sem2):
    cid = lax.axis_index("c")
    sid = lax.axis_index("s")
    w = sid * NC + cid
    base = w * ROWS_T
    pltpu.sync_copy(nid_h.at[pl.ds(base, ROWS_T)], nidv)
    for c in range(ROWS_T // KA):
        pltpu.async_copy(
            emb_h.at[nidv.at[pl.ds(c * KA, KA)]],
            rows.at[pl.ds(c * KA, KA)], sem)
    ecp0 = pltpu.async_copy(sp_h.at[pl.ds(w * ET, ET)], sidx, sem2)
    ecp1 = pltpu.async_copy(dp_h.at[pl.ds(w * ET, ET)], didx, sem2)
    for c in range(ROWS_T // KA):
        pltpu.make_async_copy(
            emb_h.at[pl.ds(0, KA)], rows.at[pl.ds(c * KA, KA)], sem).wait()
    pltpu.sync_copy(rows, erows_h.at[pl.ds(base, ROWS_T)])
    ecp0.wait()
    ecp1.wait()

    def fp(k, carry):
        fsb[pl.ds(k * 16, 16)] = jnp.zeros((16,), jnp.int32)
        fdb[pl.ds(k * 16, 16)] = jnp.broadcast_to(B + w, (16,))
        return carry
    lax.fori_loop(0, FCAP // 16, fp, jnp.int32(0))

    def comp(k, cnt):
        sv = sidx[pl.ds(k * 16, 16)]
        dv = didx[pl.ds(k * 16, 16)]
        m = dv < B
        plsc.store_compressed(fsb.at[pl.ds(cnt, 16)], sv, mask=m)
        plsc.store_compressed(fdb.at[pl.ds(cnt, 16)], dv, mask=m)
        return cnt + plsc.all_reduce_population_count(m)[0]

    cnt = lax.fori_loop(0, ET // 16, comp, jnp.int32(0))
    cbuf[...] = jnp.broadcast_to(cnt, (16,))
    pltpu.sync_copy(cbuf, fcnt_h.at[w])
    pltpu.sync_copy(fsb, fsrc_h.at[pl.ds(w * FCAP, FCAP)])
    pltpu.sync_copy(fdb, fdst_h.at[pl.ds(w * FCAP, FCAP)])


@jax.jit
def _enc_gather(emb, nid_p, sp, dp):
    return pl.kernel(
        _enc_gather_body,
        out_type=[
            jax.ShapeDtypeStruct((NPAD, C), jnp.float32),
            jax.ShapeDtypeStruct((NW * FCAP,), jnp.int32),
            jax.ShapeDtypeStruct((NW * FCAP,), jnp.int32),
            jax.ShapeDtypeStruct((NW, 16), jnp.int32),
        ],
        mesh=_MESH,
        scratch_types=[
            pltpu.VMEM((ROWS_T,), jnp.int32),
            pltpu.VMEM((ROWS_T, C), jnp.float32),
            pltpu.VMEM((ET,), jnp.int32),
            pltpu.VMEM((ET,), jnp.int32),
            pltpu.VMEM((FCAP,), jnp.int32),
            pltpu.VMEM((FCAP,), jnp.int32),
            pltpu.VMEM((16,), jnp.int32),
            pltpu.SemaphoreType.DMA,
            pltpu.SemaphoreType.DMA,
        ],
        compiler_params=_SC_PARAMS,
    )(emb, nid_p, sp, dp)


def _zero_rows(buf, nrows):
    def zr(k, carry):
        buf[k // (C // 16), pl.ds((k % (C // 16)) * 16, 16)] = (
            jnp.zeros((16,), jnp.float32))
        return carry
    lax.fori_loop(0, nrows * (C // 16), zr, jnp.int32(0))


def _msg1_body(hp_h, sp_h, dp_h, zrows_h, part_h, acc,
               sbuf0, dbuf0, sbuf1, dbuf1, rows0, rows1,
               semi0, semi1, sem0, sem1):
    cid = lax.axis_index("c")
    sid = lax.axis_index("s")
    w = sid * NC + cid
    base = w * ET
    pltpu.sync_copy(zrows_h, acc.at[pl.ds(sid * RPT, RPT)])
    plsc.subcore_barrier()

    def loadidx(c, sb, db, semi):
        off = base + c * KE
        pltpu.async_copy(sp_h.at[pl.ds(off, KE)], sb, semi)
        pltpu.async_copy(dp_h.at[pl.ds(off, KE)], db, semi)

    def waitidx(sb, db, semi):
        pltpu.make_async_copy(sp_h.at[pl.ds(0, KE)], sb, semi).wait()
        pltpu.make_async_copy(sp_h.at[pl.ds(0, KE)], db, semi).wait()

    def draingather(rows, sem):
        pltpu.make_async_copy(hp_h.at[pl.ds(0, KE)], rows, sem).wait()

    loadidx(0, sbuf0, dbuf0, semi0)
    waitidx(sbuf0, dbuf0, semi0)
    pltpu.async_copy(hp_h.at[sbuf0], rows0, sem0)
    loadidx(1, sbuf1, dbuf1, semi1)

    def body(g, carry):
        c0 = 2 * g
        waitidx(sbuf1, dbuf1, semi1)
        pltpu.async_copy(hp_h.at[sbuf1], rows1, sem1)
        draingather(rows0, sem0)
        pltpu.sync_copy(rows0, acc.at[dbuf0], add=True)

        @pl.when(c0 + 2 < NCH)
        def _():
            loadidx(c0 + 2, sbuf0, dbuf0, semi0)
            waitidx(sbuf0, dbuf0, semi0)
            pltpu.async_copy(hp_h.at[sbuf0], rows0, sem0)

        draingather(rows1, sem1)
        pltpu.sync_copy(rows1, acc.at[dbuf1], add=True)

        @pl.when(c0 + 3 < NCH)
        def _():
            loadidx(c0 + 3, sbuf1, dbuf1, semi1)
        return carry

    lax.fori_loop(0, NCH // 2, body, jnp.int32(0))
    draingather(rows0, sem0)
    pltpu.sync_copy(rows0, acc.at[dbuf0], add=True)
    plsc.subcore_barrier()
    pltpu.sync_copy(acc.at[pl.ds(sid * RPT, RPT)],
                    part_h.at[cid, pl.ds(sid * RPT, RPT)])


@jax.jit
def _msg1(hp, sp, dp, zrows):
    return pl.kernel(
        _msg1_body,
        out_type=jax.ShapeDtypeStruct((NC, NPAD, C), jnp.float32),
        mesh=_MESH,
        scratch_types=[
            pltpu.VMEM_SHARED((NPAD, C), jnp.float32),
            pltpu.VMEM((KE,), jnp.int32),
            pltpu.VMEM((KE,), jnp.int32),
            pltpu.VMEM((KE,), jnp.int32),
            pltpu.VMEM((KE,), jnp.int32),
            pltpu.VMEM((KE, C), jnp.float32),
            pltpu.VMEM((KE, C), jnp.float32),
            pltpu.SemaphoreType.DMA,
            pltpu.SemaphoreType.DMA,
            pltpu.SemaphoreType.DMA,
            pltpu.SemaphoreType.DMA,
        ],
    )(hp, sp, dp, zrows)


def _msg2_body(h1_h, fsrc_h, fdst_h, fcnt_h, z2_h, part2_h,
               acc2, sbuf, dbuf, rows, cntv, sem):
    cid = lax.axis_index("c")
    sid = lax.axis_index("s")
    w = sid * NC + cid
    pltpu.sync_copy(z2_h, acc2.at[pl.ds(sid * (A2R // NS), A2R // NS)])
    pltpu.sync_copy(fcnt_h.at[w], cntv)
    plsc.subcore_barrier()
    cnt = cntv[...][0]
    nch = (cnt + (KE2 - 1)) // KE2

    def body(c, carry):
        pltpu.sync_copy(fsrc_h.at[pl.ds(w * FCAP + c * KE2, KE2)], sbuf)
        pltpu.sync_copy(fdst_h.at[pl.ds(w * FCAP + c * KE2, KE2)], dbuf)
        pltpu.async_copy(h1_h.at[sbuf], rows, sem).wait()
        pltpu.sync_copy(rows, acc2.at[dbuf], add=True)
        return carry

    lax.fori_loop(0, nch, body, jnp.int32(0))
    plsc.subcore_barrier()
    pltpu.sync_copy(acc2.at[pl.ds(sid * (A2R // NS), A2R // NS)],
                    part2_h.at[cid, pl.ds(sid * (A2R // NS), A2R // NS)])


@jax.jit
def _msg2(h1, fsrc, fdst, fcnt, z2):
    return pl.kernel(
        _msg2_body,
        out_type=jax.ShapeDtypeStruct((NC, A2R, C), jnp.float32),
        mesh=_MESH,
        scratch_types=[
            pltpu.VMEM_SHARED((A2R, C), jnp.float32),
            pltpu.VMEM((KE2,), jnp.int32),
            pltpu.VMEM((KE2,), jnp.int32),
            pltpu.VMEM((KE2, C), jnp.float32),
            pltpu.VMEM((16,), jnp.int32),
            pltpu.SemaphoreType.DMA,
        ],
        compiler_params=_SC_PARAMS,
    )(h1, fsrc, fdst, fcnt, z2)


def _deg_body(d_ref, out_ref):
    i = pl.program_id(0)

    @pl.when(i == 0)
    def _():
        out_ref[...] = jnp.zeros_like(out_ref)

    d = d_ref[...]
    hi = d // C
    lo = d % C
    oh_hi = (hi == lax.broadcasted_iota(jnp.int32, (EB, HI), 1)
             ).astype(jnp.float32)
    oh_lo = (lo == lax.broadcasted_iota(jnp.int32, (EB, C), 1)
             ).astype(jnp.float32)
    out_ref[...] += lax.dot_general(
        oh_hi, oh_lo, (((0,), (0,)), ((), ())),
        preferred_element_type=jnp.float32)


@jax.jit
def _deg_histogram(dst2):
    return pl.pallas_call(
        _deg_body,
        grid=(EGRID,),
        in_specs=[pl.BlockSpec((EB, 1), lambda i: (i, 0))],
        out_specs=pl.BlockSpec((HI, C), lambda i: (0, 0)),
        out_shape=jax.ShapeDtypeStruct((HI, C), jnp.float32),
    )(dst2)


def _h0_body(x_ref, bv_ref, nt_ref, seed_ref, er_ref, wenc_ref, wts_ref,
             wtc_ref, benc_ref, out_ref):
    onehot = (bv_ref[...] == lax.broadcasted_iota(jnp.int32, (RB, B), 1)
              ).astype(jnp.float32)
    st = jnp.dot(onehot, seed_ref[...], preferred_element_type=jnp.float32)
    rel = st - nt_ref[...]
    k = lax.broadcasted_iota(jnp.int32, (1, C // 2), 1).astype(jnp.float32)
    freqs = jnp.exp(k * (-np.log(10000.0) / (C // 2)))
    ang = rel * freqs
    out_ref[...] = (
        jnp.dot(x_ref[...], wenc_ref[...], preferred_element_type=jnp.float32)
        + jnp.dot(jnp.sin(ang), wts_ref[...],
                  preferred_element_type=jnp.float32)
        + jnp.dot(jnp.cos(ang), wtc_ref[...],
                  preferred_element_type=jnp.float32)
        + benc_ref[...] + er_ref[...])


@jax.jit
def _h0(x_p, bv2, nt2, seed2, erows, W_enc, Wt_sin, Wt_cos, b_enc2):
    return pl.pallas_call(
        _h0_body,
        grid=(GRID,),
        in_specs=[
            pl.BlockSpec((RB, C), lambda i: (i, 0)),
            pl.BlockSpec((RB, 1), lambda i: (i, 0)),
            pl.BlockSpec((RB, 1), lambda i: (i, 0)),
            pl.BlockSpec((B, 1), lambda i: (0, 0)),
            pl.BlockSpec((RB, C), lambda i: (i, 0)),
            pl.BlockSpec((C, C), lambda i: (0, 0)),
            pl.BlockSpec((C // 2, C), lambda i: (0, 0)),
            pl.BlockSpec((C // 2, C), lambda i: (0, 0)),
            pl.BlockSpec((1, C), lambda i: (0, 0)),
        ],
        out_specs=pl.BlockSpec((RB, C), lambda i: (i, 0)),
        out_shape=jax.ShapeDtypeStruct((NPAD, C), jnp.float32),
    )(x_p, bv2, nt2, seed2, erows, W_enc, Wt_sin, Wt_cos, b_enc2)


def _layer_body(h_ref, p0_ref, p1_ref, deg_ref, ws_ref, wn_ref, bg_ref,
                out_ref):
    p = p0_ref[0] + p1_ref[0]
    deg = jnp.maximum(deg_ref[...], 1.0)
    mean = p / deg
    out_ref[...] = jnp.maximum(
        jnp.dot(h_ref[...], ws_ref[...], preferred_element_type=jnp.float32)
        + jnp.dot(mean, wn_ref[...], preferred_element_type=jnp.float32)
        + bg_ref[...], 0.0)


@jax.jit
def _layer(h, part, deg2, ws, wn, bg2):
    return pl.pallas_call(
        _layer_body,
        grid=(GRID,),
        in_specs=[
            pl.BlockSpec((RB, C), lambda i: (i, 0)),
            pl.BlockSpec((1, RB, C), lambda i: (0, i, 0)),
            pl.BlockSpec((1, RB, C), lambda i: (1, i, 0)),
            pl.BlockSpec((RB, 1), lambda i: (i, 0)),
            pl.BlockSpec((C, C), lambda i: (0, 0)),
            pl.BlockSpec((C, C), lambda i: (0, 0)),
            pl.BlockSpec((1, C), lambda i: (0, 0)),
        ],
        out_specs=pl.BlockSpec((RB, C), lambda i: (i, 0)),
        out_shape=jax.ShapeDtypeStruct((NPAD, C), jnp.float32),
    )(h, part, part, deg2, ws, wn, bg2)


def _head_body(h_ref, p0_ref, p1_ref, deg_ref, ws_ref, wn_ref, bg_ref, wh_ref,
               bh_ref, out_ref):
    p = p0_ref[0] + p1_ref[0]
    deg = jnp.maximum(deg_ref[...], 1.0)
    mean = p / deg
    h2 = jnp.maximum(
        jnp.dot(h_ref[...], ws_ref[...], preferred_element_type=jnp.float32)
        + jnp.dot(mean, wn_ref[...], preferred_element_type=jnp.float32)
        + bg_ref[...], 0.0)
    out_ref[...] = (jnp.dot(h2, wh_ref[...], preferred_element_type=jnp.float32)
                    + bh_ref[...])


@jax.jit
def _head(h, part2, deg2, ws, wn, bg2, W_head, b_head2):
    return pl.pallas_call(
        _head_body,
        grid=(1,),
        in_specs=[
            pl.BlockSpec((B, C), lambda i: (0, 0)),
            pl.BlockSpec((1, B, C), lambda i: (0, 0, 0)),
            pl.BlockSpec((1, B, C), lambda i: (1, 0, 0)),
            pl.BlockSpec((B, 1), lambda i: (0, 0)),
            pl.BlockSpec((C, C), lambda i: (0, 0)),
            pl.BlockSpec((C, C), lambda i: (0, 0)),
            pl.BlockSpec((1, C), lambda i: (0, 0)),
            pl.BlockSpec((C, 1), lambda i: (0, 0)),
            pl.BlockSpec((1, 1), lambda i: (0, 0)),
        ],
        out_specs=pl.BlockSpec((B, 1), lambda i: (0, 0)),
        out_shape=jax.ShapeDtypeStruct((B, 1), jnp.float32),
    )(h, part2, part2, deg2, ws, wn, bg2, W_head, b_head2)


def kernel(x, edge_index, n_id, node_time, seed_time, batch_vec,
           W_enc, b_enc, W_time, emb, W_self, W_neigh, b_gnn, W_head, b_head):
    pad = NPAD - N
    nid_p = jnp.concatenate([n_id, jnp.zeros((pad,), jnp.int32)])
    bv_p = jnp.concatenate([batch_vec, jnp.zeros((pad,), jnp.int32)])
    nt_p = jnp.concatenate([node_time, jnp.zeros((pad,), jnp.float32)])
    x_p = jnp.concatenate([x, jnp.zeros((pad, C), jnp.float32)])
    src = edge_index[0]
    dst = edge_index[1]
    sp = src
    dp = dst

    deg2 = _deg_histogram(dst.reshape(E, 1)).reshape(NPAD, 1)
    erows, fsrc, fdst, fcnt = _enc_gather(emb, nid_p, sp, dp)
    h0 = _h0(x_p, bv_p.reshape(NPAD, 1), nt_p.reshape(NPAD, 1),
             seed_time.reshape(B, 1), erows,
             W_enc, W_time[:C // 2], W_time[C // 2:], b_enc.reshape(1, C))
    part1 = _msg1(h0, sp, dp, jnp.zeros((RPT, C), jnp.float32))
    h1 = _layer(h0, part1, deg2, W_self[0], W_neigh[0], b_gnn[0].reshape(1, C))
    part2 = _msg2(h1, fsrc, fdst, fcnt,
                  jnp.zeros((A2R // NS, C), jnp.float32))
    return _head(h1, part2, deg2, W_self[1], W_neigh[1],
                 b_gnn[1].reshape(1, C), W_head, b_head.reshape(1, 1))


# speedup vs baseline: 1.8043x; 1.0327x over previous
import functools

import numpy as np
import jax
import jax.numpy as jnp
from jax import lax
from jax.experimental import pallas as pl
from jax.experimental.pallas import tpu as pltpu
from jax.experimental.pallas import tpu_sc as plsc

N = 10000
NPAD = 10240
E = 320000
C = 128
B = 512
NC = 2
NS = 16
NW = NC * NS
ET = E // NW
KE = 80
KE2 = 80
NCH = ET // KE
FCAP = ET + 16
A2R = 640
ROWS_T = NPAD // NW
KA = 80
RPT = NPAD // NS
RB = 512
GRID = NPAD // RB
HI = NPAD // C
EB = 6400
EGRID = E // EB

_MESH = plsc.VectorSubcoreMesh(
    core_axis_name="c", subcore_axis_name="s", num_cores=NC, num_subcores=NS)
_SC_PARAMS = pltpu.CompilerParams(needs_layout_passes=False)


def _enc_gather_body(emb_h, nid_h, sp_h, dp_h,
                     erows_h, fsrc_h, fdst_h, fcnt_h,
                     nidv, rows, sidx, didx, fsb, fdb, cbuf, sem, sem2):
    cid = lax.axis_index("c")
    sid = lax.axis_index("s")
    w = sid * NC + cid
    base = w * ROWS_T
    pltpu.sync_copy(nid_h.at[pl.ds(base, ROWS_T)], nidv)
    for c in range(ROWS_T // KA):
        pltpu.async_copy(
            emb_h.at[nidv.at[pl.ds(c * KA, KA)]],
            rows.at[pl.ds(c * KA, KA)], sem)
    ecp0 = pltpu.async_copy(sp_h.at[pl.ds(w * ET, ET)], sidx, sem2)
    ecp1 = pltpu.async_copy(dp_h.at[pl.ds(w * ET, ET)], didx, sem2)
    for c in range(ROWS_T // KA):
        pltpu.make_async_copy(
            emb_h.at[pl.ds(0, KA)], rows.at[pl.ds(c * KA, KA)], sem).wait()
    pltpu.sync_copy(rows, erows_h.at[pl.ds(base, ROWS_T)])
    ecp0.wait()
    ecp1.wait()

    def fp(k, carry):
        fsb[pl.ds(k * 16, 16)] = jnp.zeros((16,), jnp.int32)
        fdb[pl.ds(k * 16, 16)] = jnp.broadcast_to(B + w, (16,))
        return carry
    lax.fori_loop(0, FCAP // 16, fp, jnp.int32(0))

    def comp(k, cnt):
        sv = sidx[pl.ds(k * 16, 16)]
        dv = didx[pl.ds(k * 16, 16)]
        m = dv < B
        plsc.store_compressed(fsb.at[pl.ds(cnt, 16)], sv, mask=m)
        plsc.store_compressed(fdb.at[pl.ds(cnt, 16)], dv, mask=m)
        return cnt + plsc.all_reduce_population_count(m)[0]

    cnt = lax.fori_loop(0, ET // 16, comp, jnp.int32(0))
    cbuf[...] = jnp.broadcast_to(cnt, (16,))
    pltpu.sync_copy(cbuf, fcnt_h.at[w])
    pltpu.sync_copy(fsb, fsrc_h.at[pl.ds(w * FCAP, FCAP)])
    pltpu.sync_copy(fdb, fdst_h.at[pl.ds(w * FCAP, FCAP)])


@jax.jit
def _enc_gather(emb, nid_p, sp, dp):
    return pl.kernel(
        _enc_gather_body,
        out_type=[
            jax.ShapeDtypeStruct((NPAD, C), jnp.float32),
            jax.ShapeDtypeStruct((NW * FCAP,), jnp.int32),
            jax.ShapeDtypeStruct((NW * FCAP,), jnp.int32),
            jax.ShapeDtypeStruct((NW, 16), jnp.int32),
        ],
        mesh=_MESH,
        scratch_types=[
            pltpu.VMEM((ROWS_T,), jnp.int32),
            pltpu.VMEM((ROWS_T, C), jnp.float32),
            pltpu.VMEM((ET,), jnp.int32),
            pltpu.VMEM((ET,), jnp.int32),
            pltpu.VMEM((FCAP,), jnp.int32),
            pltpu.VMEM((FCAP,), jnp.int32),
            pltpu.VMEM((16,), jnp.int32),
            pltpu.SemaphoreType.DMA,
            pltpu.SemaphoreType.DMA,
        ],
        compiler_params=_SC_PARAMS,
    )(emb, nid_p, sp, dp)


def _zero_rows(buf, nrows):
    def zr(k, carry):
        buf[k // (C // 16), pl.ds((k % (C // 16)) * 16, 16)] = (
            jnp.zeros((16,), jnp.float32))
        return carry
    lax.fori_loop(0, nrows * (C // 16), zr, jnp.int32(0))


def _msg1_body(hp_h, sp_h, dp_h, zrows_h, part_h, acc,
               sbuf0, dbuf0, sbuf1, dbuf1, rows0, rows1,
               semi0, semi1, sem0, sem1):
    cid = lax.axis_index("c")
    sid = lax.axis_index("s")
    w = sid * NC + cid
    base = w * ET
    pltpu.sync_copy(zrows_h, acc.at[pl.ds(sid * RPT, RPT)])
    plsc.subcore_barrier()

    def loadidx(c, sb, db, semi):
        off = base + c * KE
        pltpu.async_copy(sp_h.at[pl.ds(off, KE)], sb, semi)
        pltpu.async_copy(dp_h.at[pl.ds(off, KE)], db, semi)

    def waitidx(sb, db, semi):
        pltpu.make_async_copy(sp_h.at[pl.ds(0, KE)], sb, semi).wait()
        pltpu.make_async_copy(sp_h.at[pl.ds(0, KE)], db, semi).wait()

    def draingather(rows, sem):
        pltpu.make_async_copy(hp_h.at[pl.ds(0, KE)], rows, sem).wait()

    loadidx(0, sbuf0, dbuf0, semi0)
    waitidx(sbuf0, dbuf0, semi0)
    pltpu.async_copy(hp_h.at[sbuf0], rows0, sem0)
    loadidx(1, sbuf1, dbuf1, semi1)

    def body(g, carry):
        c0 = 2 * g
        waitidx(sbuf1, dbuf1, semi1)
        pltpu.async_copy(hp_h.at[sbuf1], rows1, sem1)
        draingather(rows0, sem0)
        pltpu.sync_copy(rows0, acc.at[dbuf0], add=True)

        @pl.when(c0 + 2 < NCH)
        def _():
            loadidx(c0 + 2, sbuf0, dbuf0, semi0)
            waitidx(sbuf0, dbuf0, semi0)
            pltpu.async_copy(hp_h.at[sbuf0], rows0, sem0)

        draingather(rows1, sem1)
        pltpu.sync_copy(rows1, acc.at[dbuf1], add=True)

        @pl.when(c0 + 3 < NCH)
        def _():
            loadidx(c0 + 3, sbuf1, dbuf1, semi1)
        return carry

    lax.fori_loop(0, NCH // 2, body, jnp.int32(0))
    draingather(rows0, sem0)
    pltpu.sync_copy(rows0, acc.at[dbuf0], add=True)
    plsc.subcore_barrier()
    pltpu.sync_copy(acc.at[pl.ds(sid * RPT, RPT)],
                    part_h.at[cid, pl.ds(sid * RPT, RPT)])


@jax.jit
def _msg1(hp, sp, dp, zrows):
    return pl.kernel(
        _msg1_body,
        out_type=jax.ShapeDtypeStruct((NC, NPAD, C), jnp.float32),
        mesh=_MESH,
        scratch_types=[
            pltpu.VMEM_SHARED((NPAD, C), jnp.float32),
            pltpu.VMEM((KE,), jnp.int32),
            pltpu.VMEM((KE,), jnp.int32),
            pltpu.VMEM((KE,), jnp.int32),
            pltpu.VMEM((KE,), jnp.int32),
            pltpu.VMEM((KE, C), jnp.float32),
            pltpu.VMEM((KE, C), jnp.float32),
            pltpu.SemaphoreType.DMA,
            pltpu.SemaphoreType.DMA,
            pltpu.SemaphoreType.DMA,
            pltpu.SemaphoreType.DMA,
        ],
    )(hp, sp, dp, zrows)


def _msg2_body(h1_h, fsrc_h, fdst_h, fcnt_h, z2_h, part2_h,
               acc2, sbuf, dbuf, rows, cntv, sem):
    cid = lax.axis_index("c")
    sid = lax.axis_index("s")
    w = sid * NC + cid
    pltpu.sync_copy(z2_h, acc2.at[pl.ds(sid * (A2R // NS), A2R // NS)])
    pltpu.sync_copy(fcnt_h.at[w], cntv)
    plsc.subcore_barrier()
    cnt = cntv[...][0]
    nch = (cnt + (KE2 - 1)) // KE2

    def body(c, carry):
        pltpu.sync_copy(fsrc_h.at[pl.ds(w * FCAP + c * KE2, KE2)], sbuf)
        pltpu.sync_copy(fdst_h.at[pl.ds(w * FCAP + c * KE2, KE2)], dbuf)
        pltpu.async_copy(h1_h.at[sbuf], rows, sem).wait()
        pltpu.sync_copy(rows, acc2.at[dbuf], add=True)
        return carry

    lax.fori_loop(0, nch, body, jnp.int32(0))
    plsc.subcore_barrier()
    pltpu.sync_copy(acc2.at[pl.ds(sid * (A2R // NS), A2R // NS)],
                    part2_h.at[cid, pl.ds(sid * (A2R // NS), A2R // NS)])


@jax.jit
def _msg2(h1, fsrc, fdst, fcnt, z2):
    return pl.kernel(
        _msg2_body,
        out_type=jax.ShapeDtypeStruct((NC, A2R, C), jnp.float32),
        mesh=_MESH,
        scratch_types=[
            pltpu.VMEM_SHARED((A2R, C), jnp.float32),
            pltpu.VMEM((KE2,), jnp.int32),
            pltpu.VMEM((KE2,), jnp.int32),
            pltpu.VMEM((KE2, C), jnp.float32),
            pltpu.VMEM((16,), jnp.int32),
            pltpu.SemaphoreType.DMA,
        ],
        compiler_params=_SC_PARAMS,
    )(h1, fsrc, fdst, fcnt, z2)


def _deg_body(d_ref, out_ref):
    i = pl.program_id(0)

    @pl.when(i == 0)
    def _():
        out_ref[...] = jnp.zeros_like(out_ref)

    d = d_ref[...]
    hi = d // C
    lo = d % C
    oh_hi = (hi == lax.broadcasted_iota(jnp.int32, (EB, HI), 1)
             ).astype(jnp.bfloat16)
    oh_lo = (lo == lax.broadcasted_iota(jnp.int32, (EB, C), 1)
             ).astype(jnp.bfloat16)
    out_ref[...] += lax.dot_general(
        oh_hi, oh_lo, (((0,), (0,)), ((), ())),
        preferred_element_type=jnp.float32)


@jax.jit
def _deg_histogram(dst2):
    return pl.pallas_call(
        _deg_body,
        grid=(EGRID,),
        in_specs=[pl.BlockSpec((EB, 1), lambda i: (i, 0))],
        out_specs=pl.BlockSpec((HI, C), lambda i: (0, 0)),
        out_shape=jax.ShapeDtypeStruct((HI, C), jnp.float32),
    )(dst2)


def _h0_body(x_ref, bv_ref, nt_ref, seed_ref, er_ref, wenc_ref, wts_ref,
             wtc_ref, benc_ref, out_ref):
    onehot = (bv_ref[...] == lax.broadcasted_iota(jnp.int32, (RB, B), 1)
              ).astype(jnp.float32)
    st = jnp.dot(onehot, seed_ref[...], preferred_element_type=jnp.float32)
    rel = st - nt_ref[...]
    k = lax.broadcasted_iota(jnp.int32, (1, C // 2), 1).astype(jnp.float32)
    freqs = jnp.exp(k * (-np.log(10000.0) / (C // 2)))
    ang = rel * freqs
    out_ref[...] = (
        jnp.dot(x_ref[...], wenc_ref[...], preferred_element_type=jnp.float32)
        + jnp.dot(jnp.sin(ang), wts_ref[...],
                  preferred_element_type=jnp.float32)
        + jnp.dot(jnp.cos(ang), wtc_ref[...],
                  preferred_element_type=jnp.float32)
        + benc_ref[...] + er_ref[...])


@jax.jit
def _h0(x_p, bv2, nt2, seed2, erows, W_enc, Wt_sin, Wt_cos, b_enc2):
    return pl.pallas_call(
        _h0_body,
        grid=(GRID,),
        in_specs=[
            pl.BlockSpec((RB, C), lambda i: (i, 0)),
            pl.BlockSpec((RB, 1), lambda i: (i, 0)),
            pl.BlockSpec((RB, 1), lambda i: (i, 0)),
            pl.BlockSpec((B, 1), lambda i: (0, 0)),
            pl.BlockSpec((RB, C), lambda i: (i, 0)),
            pl.BlockSpec((C, C), lambda i: (0, 0)),
            pl.BlockSpec((C // 2, C), lambda i: (0, 0)),
            pl.BlockSpec((C // 2, C), lambda i: (0, 0)),
            pl.BlockSpec((1, C), lambda i: (0, 0)),
        ],
        out_specs=pl.BlockSpec((RB, C), lambda i: (i, 0)),
        out_shape=jax.ShapeDtypeStruct((NPAD, C), jnp.float32),
    )(x_p, bv2, nt2, seed2, erows, W_enc, Wt_sin, Wt_cos, b_enc2)


def _layer_body(h_ref, p0_ref, p1_ref, deg_ref, ws_ref, wn_ref, bg_ref,
                out_ref):
    p = p0_ref[0] + p1_ref[0]
    deg = jnp.maximum(deg_ref[...], 1.0)
    mean = p / deg
    out_ref[...] = jnp.maximum(
        jnp.dot(h_ref[...], ws_ref[...], preferred_element_type=jnp.float32)
        + jnp.dot(mean, wn_ref[...], preferred_element_type=jnp.float32)
        + bg_ref[...], 0.0)


@jax.jit
def _layer(h, part, deg2, ws, wn, bg2):
    return pl.pallas_call(
        _layer_body,
        grid=(GRID,),
        in_specs=[
            pl.BlockSpec((RB, C), lambda i: (i, 0)),
            pl.BlockSpec((1, RB, C), lambda i: (0, i, 0)),
            pl.BlockSpec((1, RB, C), lambda i: (1, i, 0)),
            pl.BlockSpec((RB, 1), lambda i: (i, 0)),
            pl.BlockSpec((C, C), lambda i: (0, 0)),
            pl.BlockSpec((C, C), lambda i: (0, 0)),
            pl.BlockSpec((1, C), lambda i: (0, 0)),
        ],
        out_specs=pl.BlockSpec((RB, C), lambda i: (i, 0)),
        out_shape=jax.ShapeDtypeStruct((NPAD, C), jnp.float32),
    )(h, part, part, deg2, ws, wn, bg2)


def _head_body(h_ref, p0_ref, p1_ref, deg_ref, ws_ref, wn_ref, bg_ref, wh_ref,
               bh_ref, out_ref):
    p = p0_ref[0] + p1_ref[0]
    deg = jnp.maximum(deg_ref[...], 1.0)
    mean = p / deg
    h2 = jnp.maximum(
        jnp.dot(h_ref[...], ws_ref[...], preferred_element_type=jnp.float32)
        + jnp.dot(mean, wn_ref[...], preferred_element_type=jnp.float32)
        + bg_ref[...], 0.0)
    out_ref[...] = (jnp.dot(h2, wh_ref[...], preferred_element_type=jnp.float32)
                    + bh_ref[...])


@jax.jit
def _head(h, part2, deg2, ws, wn, bg2, W_head, b_head2):
    return pl.pallas_call(
        _head_body,
        grid=(1,),
        in_specs=[
            pl.BlockSpec((B, C), lambda i: (0, 0)),
            pl.BlockSpec((1, B, C), lambda i: (0, 0, 0)),
            pl.BlockSpec((1, B, C), lambda i: (1, 0, 0)),
            pl.BlockSpec((B, 1), lambda i: (0, 0)),
            pl.BlockSpec((C, C), lambda i: (0, 0)),
            pl.BlockSpec((C, C), lambda i: (0, 0)),
            pl.BlockSpec((1, C), lambda i: (0, 0)),
            pl.BlockSpec((C, 1), lambda i: (0, 0)),
            pl.BlockSpec((1, 1), lambda i: (0, 0)),
        ],
        out_specs=pl.BlockSpec((B, 1), lambda i: (0, 0)),
        out_shape=jax.ShapeDtypeStruct((B, 1), jnp.float32),
    )(h, part2, part2, deg2, ws, wn, bg2, W_head, b_head2)


def kernel(x, edge_index, n_id, node_time, seed_time, batch_vec,
           W_enc, b_enc, W_time, emb, W_self, W_neigh, b_gnn, W_head, b_head):
    pad = NPAD - N
    nid_p = jnp.concatenate([n_id, jnp.zeros((pad,), jnp.int32)])
    bv_p = jnp.concatenate([batch_vec, jnp.zeros((pad,), jnp.int32)])
    nt_p = jnp.concatenate([node_time, jnp.zeros((pad,), jnp.float32)])
    x_p = jnp.concatenate([x, jnp.zeros((pad, C), jnp.float32)])
    src = edge_index[0]
    dst = edge_index[1]
    sp = src
    dp = dst

    erows, fsrc, fdst, fcnt = _enc_gather(emb, nid_p, sp, dp)
    h0 = _h0(x_p, bv_p.reshape(NPAD, 1), nt_p.reshape(NPAD, 1),
             seed_time.reshape(B, 1), erows,
             W_enc, W_time[:C // 2], W_time[C // 2:], b_enc.reshape(1, C))
    part1 = _msg1(h0, sp, dp, jnp.zeros((RPT, C), jnp.float32))
    deg2 = _deg_histogram(dst.reshape(E, 1)).reshape(NPAD, 1)
    h1 = _layer(h0, part1, deg2, W_self[0], W_neigh[0], b_gnn[0].reshape(1, C))
    part2 = _msg2(h1, fsrc, fdst, fcnt,
                  jnp.zeros((A2R // NS, C), jnp.float32))
    return _head(h1, part2, deg2, W_self[1], W_neigh[1],
                 b_gnn[1].reshape(1, C), W_head, b_head.reshape(1, 1))


# speedup vs baseline: 1.8312x; 1.0149x over previous
import functools

import numpy as np
import jax
import jax.numpy as jnp
from jax import lax
from jax.experimental import pallas as pl
from jax.experimental.pallas import tpu as pltpu
from jax.experimental.pallas import tpu_sc as plsc

N = 10000
NPAD = 10240
E = 320000
C = 128
B = 512
NC = 2
NS = 16
NW = NC * NS
ET = E // NW
KE = 80
KE2 = 80
NCH = ET // KE
FCAP = ET + 16
A2R = 640
ROWS_T = NPAD // NW
KA = 80
RPT = NPAD // NS
RB = 512
GRID = NPAD // RB
HI = NPAD // C
EB = 6400
EGRID = E // EB

_MESH = plsc.VectorSubcoreMesh(
    core_axis_name="c", subcore_axis_name="s", num_cores=NC, num_subcores=NS)
_SC_PARAMS = pltpu.CompilerParams(needs_layout_passes=False)


def _enc_gather_body(emb_h, nid_h, sp_h, dp_h,
                     erows_h, fsrc_h, fdst_h, fcnt_h,
                     nidv, rows, sidx, didx, fsb, fdb, cbuf, sem, sem2):
    cid = lax.axis_index("c")
    sid = lax.axis_index("s")
    w = sid * NC + cid
    base = w * ROWS_T
    pltpu.sync_copy(nid_h.at[pl.ds(base, ROWS_T)], nidv)
    for c in range(ROWS_T // KA):
        pltpu.async_copy(
            emb_h.at[nidv.at[pl.ds(c * KA, KA)]],
            rows.at[pl.ds(c * KA, KA)], sem)
    ecp0 = pltpu.async_copy(sp_h.at[pl.ds(w * ET, ET)], sidx, sem2)
    ecp1 = pltpu.async_copy(dp_h.at[pl.ds(w * ET, ET)], didx, sem2)
    for c in range(ROWS_T // KA):
        pltpu.make_async_copy(
            emb_h.at[pl.ds(0, KA)], rows.at[pl.ds(c * KA, KA)], sem).wait()
    pltpu.sync_copy(rows, erows_h.at[pl.ds(base, ROWS_T)])
    ecp0.wait()
    ecp1.wait()

    def fp(k, carry):
        fsb[pl.ds(k * 16, 16)] = jnp.zeros((16,), jnp.int32)
        fdb[pl.ds(k * 16, 16)] = jnp.broadcast_to(B + w, (16,))
        return carry
    lax.fori_loop(0, FCAP // 16, fp, jnp.int32(0))

    def comp(k, cnt):
        sv = sidx[pl.ds(k * 16, 16)]
        dv = didx[pl.ds(k * 16, 16)]
        m = dv < B
        plsc.store_compressed(fsb.at[pl.ds(cnt, 16)], sv, mask=m)
        plsc.store_compressed(fdb.at[pl.ds(cnt, 16)], dv, mask=m)
        return cnt + plsc.all_reduce_population_count(m)[0]

    cnt = lax.fori_loop(0, ET // 16, comp, jnp.int32(0))
    cbuf[...] = jnp.broadcast_to(cnt, (16,))
    pltpu.sync_copy(cbuf, fcnt_h.at[w])
    pltpu.sync_copy(fsb, fsrc_h.at[pl.ds(w * FCAP, FCAP)])
    pltpu.sync_copy(fdb, fdst_h.at[pl.ds(w * FCAP, FCAP)])


@jax.jit
def _enc_gather(emb, nid_p, sp, dp):
    return pl.kernel(
        _enc_gather_body,
        out_type=[
            jax.ShapeDtypeStruct((NPAD, C), jnp.float32),
            jax.ShapeDtypeStruct((NW * FCAP,), jnp.int32),
            jax.ShapeDtypeStruct((NW * FCAP,), jnp.int32),
            jax.ShapeDtypeStruct((NW, 16), jnp.int32),
        ],
        mesh=_MESH,
        scratch_types=[
            pltpu.VMEM((ROWS_T,), jnp.int32),
            pltpu.VMEM((ROWS_T, C), jnp.float32),
            pltpu.VMEM((ET,), jnp.int32),
            pltpu.VMEM((ET,), jnp.int32),
            pltpu.VMEM((FCAP,), jnp.int32),
            pltpu.VMEM((FCAP,), jnp.int32),
            pltpu.VMEM((16,), jnp.int32),
            pltpu.SemaphoreType.DMA,
            pltpu.SemaphoreType.DMA,
        ],
        compiler_params=_SC_PARAMS,
    )(emb, nid_p, sp, dp)


def _zero_rows(buf, nrows):
    def zr(k, carry):
        buf[k // (C // 16), pl.ds((k % (C // 16)) * 16, 16)] = (
            jnp.zeros((16,), jnp.float32))
        return carry
    lax.fori_loop(0, nrows * (C // 16), zr, jnp.int32(0))


def _msg1_body(hp_h, sp_h, dp_h, zrows_h, part_h, acc,
               sbuf0, dbuf0, sbuf1, dbuf1, rows0, rows1,
               semi0, semi1, sem0, sem1):
    cid = lax.axis_index("c")
    sid = lax.axis_index("s")
    w = sid * NC + cid
    base = w * ET
    pltpu.sync_copy(zrows_h, acc.at[pl.ds(sid * RPT, RPT)])
    plsc.subcore_barrier()

    def loadidx(c, sb, db, semi):
        off = base + c * KE
        pltpu.async_copy(sp_h.at[pl.ds(off, KE)], sb, semi)
        pltpu.async_copy(dp_h.at[pl.ds(off, KE)], db, semi)

    def waitidx(sb, db, semi):
        pltpu.make_async_copy(sp_h.at[pl.ds(0, KE)], sb, semi).wait()
        pltpu.make_async_copy(sp_h.at[pl.ds(0, KE)], db, semi).wait()

    def draingather(rows, sem):
        pltpu.make_async_copy(hp_h.at[pl.ds(0, KE)], rows, sem).wait()

    loadidx(0, sbuf0, dbuf0, semi0)
    waitidx(sbuf0, dbuf0, semi0)
    pltpu.async_copy(hp_h.at[sbuf0], rows0, sem0)
    loadidx(1, sbuf1, dbuf1, semi1)

    def body(g, carry):
        c0 = 2 * g
        waitidx(sbuf1, dbuf1, semi1)
        pltpu.async_copy(hp_h.at[sbuf1], rows1, sem1)
        draingather(rows0, sem0)
        pltpu.sync_copy(rows0, acc.at[dbuf0], add=True)

        @pl.when(c0 + 2 < NCH)
        def _():
            loadidx(c0 + 2, sbuf0, dbuf0, semi0)
            waitidx(sbuf0, dbuf0, semi0)
            pltpu.async_copy(hp_h.at[sbuf0], rows0, sem0)

        draingather(rows1, sem1)
        pltpu.sync_copy(rows1, acc.at[dbuf1], add=True)

        @pl.when(c0 + 3 < NCH)
        def _():
            loadidx(c0 + 3, sbuf1, dbuf1, semi1)
        return carry

    lax.fori_loop(0, NCH // 2, body, jnp.int32(0))
    draingather(rows0, sem0)
    pltpu.sync_copy(rows0, acc.at[dbuf0], add=True)
    plsc.subcore_barrier()
    pltpu.sync_copy(acc.at[pl.ds(sid * RPT, RPT)],
                    part_h.at[cid, pl.ds(sid * RPT, RPT)])


@jax.jit
def _msg1(hp, sp, dp, zrows):
    return pl.kernel(
        _msg1_body,
        out_type=jax.ShapeDtypeStruct((NC, NPAD, C), jnp.float32),
        mesh=_MESH,
        scratch_types=[
            pltpu.VMEM_SHARED((NPAD, C), jnp.float32),
            pltpu.VMEM((KE,), jnp.int32),
            pltpu.VMEM((KE,), jnp.int32),
            pltpu.VMEM((KE,), jnp.int32),
            pltpu.VMEM((KE,), jnp.int32),
            pltpu.VMEM((KE, C), jnp.float32),
            pltpu.VMEM((KE, C), jnp.float32),
            pltpu.SemaphoreType.DMA,
            pltpu.SemaphoreType.DMA,
            pltpu.SemaphoreType.DMA,
            pltpu.SemaphoreType.DMA,
        ],
    )(hp, sp, dp, zrows)


def _msg2_body(h1_h, fsrc_h, fdst_h, fcnt_h, z2_h, part2_h,
               acc2, sbuf0, dbuf0, sbuf1, dbuf1, rows0, rows1, cntv,
               semi0, semi1, sem0, sem1):
    cid = lax.axis_index("c")
    sid = lax.axis_index("s")
    w = sid * NC + cid
    pltpu.sync_copy(z2_h, acc2.at[pl.ds(sid * (A2R // NS), A2R // NS)])
    pltpu.sync_copy(fcnt_h.at[w], cntv)
    plsc.subcore_barrier()
    cnt = cntv[...][0]
    nch = (cnt + (KE2 - 1)) // KE2

    def loadidx(c, sb, db, semi):
        off = w * FCAP + c * KE2
        pltpu.async_copy(fsrc_h.at[pl.ds(off, KE2)], sb, semi)
        pltpu.async_copy(fdst_h.at[pl.ds(off, KE2)], db, semi)

    def waitidx(sb, db, semi):
        pltpu.make_async_copy(fsrc_h.at[pl.ds(0, KE2)], sb, semi).wait()
        pltpu.make_async_copy(fsrc_h.at[pl.ds(0, KE2)], db, semi).wait()

    def draingather(rows, sem):
        pltpu.make_async_copy(h1_h.at[pl.ds(0, KE2)], rows, sem).wait()

    @pl.when(nch > 0)
    def _():
        loadidx(0, sbuf0, dbuf0, semi0)
        waitidx(sbuf0, dbuf0, semi0)
        pltpu.async_copy(h1_h.at[sbuf0], rows0, sem0)

        @pl.when(nch > 1)
        def _():
            loadidx(1, sbuf1, dbuf1, semi1)

        def body(g, carry):
            c0 = 2 * g

            @pl.when(c0 + 1 < nch)
            def _():
                waitidx(sbuf1, dbuf1, semi1)
                pltpu.async_copy(h1_h.at[sbuf1], rows1, sem1)

            draingather(rows0, sem0)
            pltpu.sync_copy(rows0, acc2.at[dbuf0], add=True)

            @pl.when(c0 + 2 < nch)
            def _():
                loadidx(c0 + 2, sbuf0, dbuf0, semi0)
                waitidx(sbuf0, dbuf0, semi0)
                pltpu.async_copy(h1_h.at[sbuf0], rows0, sem0)

            @pl.when(c0 + 1 < nch)
            def _():
                draingather(rows1, sem1)
                pltpu.sync_copy(rows1, acc2.at[dbuf1], add=True)

                @pl.when(c0 + 3 < nch)
                def _():
                    loadidx(c0 + 3, sbuf1, dbuf1, semi1)
            return carry

        lax.fori_loop(0, (nch + 1) // 2, body, jnp.int32(0))

    plsc.subcore_barrier()
    pltpu.sync_copy(acc2.at[pl.ds(sid * (A2R // NS), A2R // NS)],
                    part2_h.at[cid, pl.ds(sid * (A2R // NS), A2R // NS)])


@jax.jit
def _msg2(h1, fsrc, fdst, fcnt, z2):
    return pl.kernel(
        _msg2_body,
        out_type=jax.ShapeDtypeStruct((NC, A2R, C), jnp.float32),
        mesh=_MESH,
        scratch_types=[
            pltpu.VMEM_SHARED((A2R, C), jnp.float32),
            pltpu.VMEM((KE2,), jnp.int32),
            pltpu.VMEM((KE2,), jnp.int32),
            pltpu.VMEM((KE2,), jnp.int32),
            pltpu.VMEM((KE2,), jnp.int32),
            pltpu.VMEM((KE2, C), jnp.float32),
            pltpu.VMEM((KE2, C), jnp.float32),
            pltpu.VMEM((16,), jnp.int32),
            pltpu.SemaphoreType.DMA,
            pltpu.SemaphoreType.DMA,
            pltpu.SemaphoreType.DMA,
            pltpu.SemaphoreType.DMA,
        ],
        compiler_params=_SC_PARAMS,
    )(h1, fsrc, fdst, fcnt, z2)


def _deg_body(d_ref, out_ref):
    i = pl.program_id(0)

    @pl.when(i == 0)
    def _():
        out_ref[...] = jnp.zeros_like(out_ref)

    d = d_ref[...]
    hi = d // C
    lo = d % C
    oh_hi = (hi == lax.broadcasted_iota(jnp.int32, (EB, HI), 1)
             ).astype(jnp.bfloat16)
    oh_lo = (lo == lax.broadcasted_iota(jnp.int32, (EB, C), 1)
             ).astype(jnp.bfloat16)
    out_ref[...] += lax.dot_general(
        oh_hi, oh_lo, (((0,), (0,)), ((), ())),
        preferred_element_type=jnp.float32)


@jax.jit
def _deg_histogram(dst2):
    return pl.pallas_call(
        _deg_body,
        grid=(EGRID,),
        in_specs=[pl.BlockSpec((EB, 1), lambda i: (i, 0))],
        out_specs=pl.BlockSpec((HI, C), lambda i: (0, 0)),
        out_shape=jax.ShapeDtypeStruct((HI, C), jnp.float32),
    )(dst2)


def _h0_body(x_ref, bv_ref, nt_ref, seed_ref, er_ref, wenc_ref, wts_ref,
             wtc_ref, benc_ref, out_ref):
    onehot = (bv_ref[...] == lax.broadcasted_iota(jnp.int32, (RB, B), 1)
              ).astype(jnp.float32)
    st = jnp.dot(onehot, seed_ref[...], preferred_element_type=jnp.float32)
    rel = st - nt_ref[...]
    k = lax.broadcasted_iota(jnp.int32, (1, C // 2), 1).astype(jnp.float32)
    freqs = jnp.exp(k * (-np.log(10000.0) / (C // 2)))
    ang = rel * freqs
    out_ref[...] = (
        jnp.dot(x_ref[...], wenc_ref[...], preferred_element_type=jnp.float32)
        + jnp.dot(jnp.sin(ang), wts_ref[...],
                  preferred_element_type=jnp.float32)
        + jnp.dot(jnp.cos(ang), wtc_ref[...],
                  preferred_element_type=jnp.float32)
        + benc_ref[...] + er_ref[...])


@jax.jit
def _h0(x_p, bv2, nt2, seed2, erows, W_enc, Wt_sin, Wt_cos, b_enc2):
    return pl.pallas_call(
        _h0_body,
        grid=(GRID,),
        in_specs=[
            pl.BlockSpec((RB, C), lambda i: (i, 0)),
            pl.BlockSpec((RB, 1), lambda i: (i, 0)),
            pl.BlockSpec((RB, 1), lambda i: (i, 0)),
            pl.BlockSpec((B, 1), lambda i: (0, 0)),
            pl.BlockSpec((RB, C), lambda i: (i, 0)),
            pl.BlockSpec((C, C), lambda i: (0, 0)),
            pl.BlockSpec((C // 2, C), lambda i: (0, 0)),
            pl.BlockSpec((C // 2, C), lambda i: (0, 0)),
            pl.BlockSpec((1, C), lambda i: (0, 0)),
        ],
        out_specs=pl.BlockSpec((RB, C), lambda i: (i, 0)),
        out_shape=jax.ShapeDtypeStruct((NPAD, C), jnp.float32),
    )(x_p, bv2, nt2, seed2, erows, W_enc, Wt_sin, Wt_cos, b_enc2)


def _layer_body(h_ref, p0_ref, p1_ref, deg_ref, ws_ref, wn_ref, bg_ref,
                out_ref):
    p = p0_ref[0] + p1_ref[0]
    deg = jnp.maximum(deg_ref[...], 1.0)
    mean = p / deg
    out_ref[...] = jnp.maximum(
        jnp.dot(h_ref[...], ws_ref[...], preferred_element_type=jnp.float32)
        + jnp.dot(mean, wn_ref[...], preferred_element_type=jnp.float32)
        + bg_ref[...], 0.0)


@jax.jit
def _layer(h, part, deg2, ws, wn, bg2):
    return pl.pallas_call(
        _layer_body,
        grid=(GRID,),
        in_specs=[
            pl.BlockSpec((RB, C), lambda i: (i, 0)),
            pl.BlockSpec((1, RB, C), lambda i: (0, i, 0)),
            pl.BlockSpec((1, RB, C), lambda i: (1, i, 0)),
            pl.BlockSpec((RB, 1), lambda i: (i, 0)),
            pl.BlockSpec((C, C), lambda i: (0, 0)),
            pl.BlockSpec((C, C), lambda i: (0, 0)),
            pl.BlockSpec((1, C), lambda i: (0, 0)),
        ],
        out_specs=pl.BlockSpec((RB, C), lambda i: (i, 0)),
        out_shape=jax.ShapeDtypeStruct((NPAD, C), jnp.float32),
    )(h, part, part, deg2, ws, wn, bg2)


def _head_body(h_ref, p0_ref, p1_ref, deg_ref, ws_ref, wn_ref, bg_ref, wh_ref,
               bh_ref, out_ref):
    p = p0_ref[0] + p1_ref[0]
    deg = jnp.maximum(deg_ref[...], 1.0)
    mean = p / deg
    h2 = jnp.maximum(
        jnp.dot(h_ref[...], ws_ref[...], preferred_element_type=jnp.float32)
        + jnp.dot(mean, wn_ref[...], preferred_element_type=jnp.float32)
        + bg_ref[...], 0.0)
    out_ref[...] = (jnp.dot(h2, wh_ref[...], preferred_element_type=jnp.float32)
                    + bh_ref[...])


@jax.jit
def _head(h, part2, deg2, ws, wn, bg2, W_head, b_head2):
    return pl.pallas_call(
        _head_body,
        grid=(1,),
        in_specs=[
            pl.BlockSpec((B, C), lambda i: (0, 0)),
            pl.BlockSpec((1, B, C), lambda i: (0, 0, 0)),
            pl.BlockSpec((1, B, C), lambda i: (1, 0, 0)),
            pl.BlockSpec((B, 1), lambda i: (0, 0)),
            pl.BlockSpec((C, C), lambda i: (0, 0)),
            pl.BlockSpec((C, C), lambda i: (0, 0)),
            pl.BlockSpec((1, C), lambda i: (0, 0)),
            pl.BlockSpec((C, 1), lambda i: (0, 0)),
            pl.BlockSpec((1, 1), lambda i: (0, 0)),
        ],
        out_specs=pl.BlockSpec((B, 1), lambda i: (0, 0)),
        out_shape=jax.ShapeDtypeStruct((B, 1), jnp.float32),
    )(h, part2, part2, deg2, ws, wn, bg2, W_head, b_head2)


def kernel(x, edge_index, n_id, node_time, seed_time, batch_vec,
           W_enc, b_enc, W_time, emb, W_self, W_neigh, b_gnn, W_head, b_head):
    pad = NPAD - N
    nid_p = jnp.concatenate([n_id, jnp.zeros((pad,), jnp.int32)])
    bv_p = jnp.concatenate([batch_vec, jnp.zeros((pad,), jnp.int32)])
    nt_p = jnp.concatenate([node_time, jnp.zeros((pad,), jnp.float32)])
    x_p = jnp.concatenate([x, jnp.zeros((pad, C), jnp.float32)])
    src = edge_index[0]
    dst = edge_index[1]
    sp = src
    dp = dst

    erows, fsrc, fdst, fcnt = _enc_gather(emb, nid_p, sp, dp)
    h0 = _h0(x_p, bv_p.reshape(NPAD, 1), nt_p.reshape(NPAD, 1),
             seed_time.reshape(B, 1), erows,
             W_enc, W_time[:C // 2], W_time[C // 2:], b_enc.reshape(1, C))
    part1 = _msg1(h0, sp, dp, jnp.zeros((RPT, C), jnp.float32))
    deg2 = _deg_histogram(dst.reshape(E, 1)).reshape(NPAD, 1)
    h1 = _layer(h0, part1, deg2, W_self[0], W_neigh[0], b_gnn[0].reshape(1, C))
    part2 = _msg2(h1, fsrc, fdst, fcnt,
                  jnp.zeros((A2R // NS, C), jnp.float32))
    return _head(h1, part2, deg2, W_self[1], W_neigh[1],
                 b_gnn[1].reshape(1, C), W_head, b_head.reshape(1, 1))


# speedup vs baseline: 1.8319x; 1.0004x over previous
import numpy as np
import jax
import jax.numpy as jnp
from jax import lax
from jax.experimental import pallas as pl
from jax.experimental.pallas import tpu as pltpu
from jax.experimental.pallas import tpu_sc as plsc

N = 10000
NPAD = 10240
E = 320000
C = 128
B = 512
NC = 2
NS = 16
NW = NC * NS
ET = E // NW
KE = 80
KE2 = 80
NCH = ET // KE
FCAP = ET + 16
A2R = 640
ROWS_T = NPAD // NW
KA = 80
RPT = NPAD // NS
RB = 512
GRID = NPAD // RB
HI = NPAD // C
EB = 6400
EGRID = E // EB

_MESH = plsc.VectorSubcoreMesh(
    core_axis_name="c", subcore_axis_name="s", num_cores=NC, num_subcores=NS)
_SC_PARAMS = pltpu.CompilerParams(needs_layout_passes=False)


def _enc_gather_body(emb_h, nid_h, sp_h, dp_h,
                     erows_h, fsrc_h, fdst_h, fcnt_h,
                     nidv, rows, sidx, didx, fsb, fdb, cbuf, sem, sem2):
    cid = lax.axis_index("c")
    sid = lax.axis_index("s")
    w = sid * NC + cid
    base = w * ROWS_T
    pltpu.sync_copy(nid_h.at[pl.ds(base, ROWS_T)], nidv)
    for c in range(ROWS_T // KA):
        pltpu.async_copy(
            emb_h.at[nidv.at[pl.ds(c * KA, KA)]],
            rows.at[pl.ds(c * KA, KA)], sem)
    ecp0 = pltpu.async_copy(sp_h.at[pl.ds(w * ET, ET)], sidx, sem2)
    ecp1 = pltpu.async_copy(dp_h.at[pl.ds(w * ET, ET)], didx, sem2)
    for c in range(ROWS_T // KA):
        pltpu.make_async_copy(
            emb_h.at[pl.ds(0, KA)], rows.at[pl.ds(c * KA, KA)], sem).wait()
    pltpu.sync_copy(rows, erows_h.at[pl.ds(base, ROWS_T)])
    ecp0.wait()
    ecp1.wait()

    def fp(k, carry):
        fsb[pl.ds(k * 16, 16)] = jnp.zeros((16,), jnp.int32)
        fdb[pl.ds(k * 16, 16)] = jnp.broadcast_to(B + w, (16,))
        return carry
    lax.fori_loop(0, FCAP // 16, fp, jnp.int32(0))

    def comp(k, cnt):
        sv = sidx[pl.ds(k * 16, 16)]
        dv = didx[pl.ds(k * 16, 16)]
        m = dv < B
        plsc.store_compressed(fsb.at[pl.ds(cnt, 16)], sv, mask=m)
        plsc.store_compressed(fdb.at[pl.ds(cnt, 16)], dv, mask=m)
        return cnt + plsc.all_reduce_population_count(m)[0]

    cnt = lax.fori_loop(0, ET // 16, comp, jnp.int32(0))
    cbuf[...] = jnp.broadcast_to(cnt, (16,))
    pltpu.sync_copy(cbuf, fcnt_h.at[w])
    pltpu.sync_copy(fsb, fsrc_h.at[pl.ds(w * FCAP, FCAP)])
    pltpu.sync_copy(fdb, fdst_h.at[pl.ds(w * FCAP, FCAP)])


@jax.jit
def _enc_gather(emb, nid_p, sp, dp):
    return pl.kernel(
        _enc_gather_body,
        out_type=[
            jax.ShapeDtypeStruct((NPAD, C), jnp.float32),
            jax.ShapeDtypeStruct((NW * FCAP,), jnp.int32),
            jax.ShapeDtypeStruct((NW * FCAP,), jnp.int32),
            jax.ShapeDtypeStruct((NW, 16), jnp.int32),
        ],
        mesh=_MESH,
        scratch_types=[
            pltpu.VMEM((ROWS_T,), jnp.int32),
            pltpu.VMEM((ROWS_T, C), jnp.float32),
            pltpu.VMEM((ET,), jnp.int32),
            pltpu.VMEM((ET,), jnp.int32),
            pltpu.VMEM((FCAP,), jnp.int32),
            pltpu.VMEM((FCAP,), jnp.int32),
            pltpu.VMEM((16,), jnp.int32),
            pltpu.SemaphoreType.DMA,
            pltpu.SemaphoreType.DMA,
        ],
        compiler_params=_SC_PARAMS,
    )(emb, nid_p, sp, dp)


def _msg1_body(hp_h, sp_h, dp_h, zrows_h, part_h, acc,
               sbuf0, dbuf0, sbuf1, dbuf1, rows0, rows1,
               semi0, semi1, sem0, sem1):
    cid = lax.axis_index("c")
    sid = lax.axis_index("s")
    w = sid * NC + cid
    base = w * ET
    pltpu.sync_copy(zrows_h, acc.at[pl.ds(sid * RPT, RPT)])
    plsc.subcore_barrier()

    def loadidx(c, sb, db, semi):
        off = base + c * KE
        pltpu.async_copy(sp_h.at[pl.ds(off, KE)], sb, semi)
        pltpu.async_copy(dp_h.at[pl.ds(off, KE)], db, semi)

    def waitidx(sb, db, semi):
        pltpu.make_async_copy(sp_h.at[pl.ds(0, KE)], sb, semi).wait()
        pltpu.make_async_copy(sp_h.at[pl.ds(0, KE)], db, semi).wait()

    def draingather(rows, sem):
        pltpu.make_async_copy(hp_h.at[pl.ds(0, KE)], rows, sem).wait()

    loadidx(0, sbuf0, dbuf0, semi0)
    waitidx(sbuf0, dbuf0, semi0)
    pltpu.async_copy(hp_h.at[sbuf0], rows0, sem0)
    loadidx(1, sbuf1, dbuf1, semi1)

    def body(g, carry):
        c0 = 2 * g
        waitidx(sbuf1, dbuf1, semi1)
        pltpu.async_copy(hp_h.at[sbuf1], rows1, sem1)
        draingather(rows0, sem0)
        pltpu.sync_copy(rows0, acc.at[dbuf0], add=True)

        @pl.when(c0 + 2 < NCH)
        def _():
            loadidx(c0 + 2, sbuf0, dbuf0, semi0)
            waitidx(sbuf0, dbuf0, semi0)
            pltpu.async_copy(hp_h.at[sbuf0], rows0, sem0)

        draingather(rows1, sem1)
        pltpu.sync_copy(rows1, acc.at[dbuf1], add=True)

        @pl.when(c0 + 3 < NCH)
        def _():
            loadidx(c0 + 3, sbuf1, dbuf1, semi1)
        return carry

    lax.fori_loop(0, NCH // 2, body, jnp.int32(0))
    draingather(rows0, sem0)
    pltpu.sync_copy(rows0, acc.at[dbuf0], add=True)
    plsc.subcore_barrier()
    pltpu.sync_copy(acc.at[pl.ds(sid * RPT, RPT)],
                    part_h.at[cid, pl.ds(sid * RPT, RPT)])


@jax.jit
def _msg1(hp, sp, dp, zrows):
    return pl.kernel(
        _msg1_body,
        out_type=jax.ShapeDtypeStruct((NC, NPAD, C), jnp.float32),
        mesh=_MESH,
        scratch_types=[
            pltpu.VMEM_SHARED((NPAD, C), jnp.float32),
            pltpu.VMEM((KE,), jnp.int32),
            pltpu.VMEM((KE,), jnp.int32),
            pltpu.VMEM((KE,), jnp.int32),
            pltpu.VMEM((KE,), jnp.int32),
            pltpu.VMEM((KE, C), jnp.float32),
            pltpu.VMEM((KE, C), jnp.float32),
            pltpu.SemaphoreType.DMA,
            pltpu.SemaphoreType.DMA,
            pltpu.SemaphoreType.DMA,
            pltpu.SemaphoreType.DMA,
        ],
    )(hp, sp, dp, zrows)


def _msg2_body(h1_h, fsrc_h, fdst_h, fcnt_h, z2_h, part2_h,
               acc2, sbuf0, dbuf0, sbuf1, dbuf1, rows0, rows1, cntv,
               semi0, semi1, sem0, sem1):
    cid = lax.axis_index("c")
    sid = lax.axis_index("s")
    w = sid * NC + cid
    pltpu.sync_copy(z2_h, acc2.at[pl.ds(sid * (A2R // NS), A2R // NS)])
    pltpu.sync_copy(fcnt_h.at[w], cntv)
    plsc.subcore_barrier()
    cnt = cntv[...][0]
    nch = (cnt + (KE2 - 1)) // KE2

    def loadidx(c, sb, db, semi):
        off = w * FCAP + c * KE2
        pltpu.async_copy(fsrc_h.at[pl.ds(off, KE2)], sb, semi)
        pltpu.async_copy(fdst_h.at[pl.ds(off, KE2)], db, semi)

    def waitidx(sb, db, semi):
        pltpu.make_async_copy(fsrc_h.at[pl.ds(0, KE2)], sb, semi).wait()
        pltpu.make_async_copy(fsrc_h.at[pl.ds(0, KE2)], db, semi).wait()

    def draingather(rows, sem):
        pltpu.make_async_copy(h1_h.at[pl.ds(0, KE2)], rows, sem).wait()

    @pl.when(nch > 0)
    def _():
        loadidx(0, sbuf0, dbuf0, semi0)
        waitidx(sbuf0, dbuf0, semi0)
        pltpu.async_copy(h1_h.at[sbuf0], rows0, sem0)

        @pl.when(nch > 1)
        def _():
            loadidx(1, sbuf1, dbuf1, semi1)

        def body(g, carry):
            c0 = 2 * g

            @pl.when(c0 + 1 < nch)
            def _():
                waitidx(sbuf1, dbuf1, semi1)
                pltpu.async_copy(h1_h.at[sbuf1], rows1, sem1)

            draingather(rows0, sem0)
            pltpu.sync_copy(rows0, acc2.at[dbuf0], add=True)

            @pl.when(c0 + 2 < nch)
            def _():
                loadidx(c0 + 2, sbuf0, dbuf0, semi0)
                waitidx(sbuf0, dbuf0, semi0)
                pltpu.async_copy(h1_h.at[sbuf0], rows0, sem0)

            @pl.when(c0 + 1 < nch)
            def _():
                draingather(rows1, sem1)
                pltpu.sync_copy(rows1, acc2.at[dbuf1], add=True)

                @pl.when(c0 + 3 < nch)
                def _():
                    loadidx(c0 + 3, sbuf1, dbuf1, semi1)
            return carry

        lax.fori_loop(0, (nch + 1) // 2, body, jnp.int32(0))

    plsc.subcore_barrier()
    pltpu.sync_copy(acc2.at[pl.ds(sid * (A2R // NS), A2R // NS)],
                    part2_h.at[cid, pl.ds(sid * (A2R // NS), A2R // NS)])


@jax.jit
def _msg2(h1, fsrc, fdst, fcnt, z2):
    return pl.kernel(
        _msg2_body,
        out_type=jax.ShapeDtypeStruct((NC, A2R, C), jnp.float32),
        mesh=_MESH,
        scratch_types=[
            pltpu.VMEM_SHARED((A2R, C), jnp.float32),
            pltpu.VMEM((KE2,), jnp.int32),
            pltpu.VMEM((KE2,), jnp.int32),
            pltpu.VMEM((KE2,), jnp.int32),
            pltpu.VMEM((KE2,), jnp.int32),
            pltpu.VMEM((KE2, C), jnp.float32),
            pltpu.VMEM((KE2, C), jnp.float32),
            pltpu.VMEM((16,), jnp.int32),
            pltpu.SemaphoreType.DMA,
            pltpu.SemaphoreType.DMA,
            pltpu.SemaphoreType.DMA,
            pltpu.SemaphoreType.DMA,
        ],
        compiler_params=_SC_PARAMS,
    )(h1, fsrc, fdst, fcnt, z2)


def _deg_body(d_ref, out_ref):
    i = pl.program_id(0)

    @pl.when(i == 0)
    def _():
        out_ref[...] = jnp.zeros_like(out_ref)

    d = d_ref[...]
    hi = d // C
    lo = d % C
    oh_hi = (hi == lax.broadcasted_iota(jnp.int32, (EB, HI), 1)
             ).astype(jnp.bfloat16)
    oh_lo = (lo == lax.broadcasted_iota(jnp.int32, (EB, C), 1)
             ).astype(jnp.bfloat16)
    out_ref[...] += lax.dot_general(
        oh_hi, oh_lo, (((0,), (0,)), ((), ())),
        preferred_element_type=jnp.float32)


@jax.jit
def _deg_histogram(dst2):
    return pl.pallas_call(
        _deg_body,
        grid=(EGRID,),
        in_specs=[pl.BlockSpec((EB, 1), lambda i: (i, 0))],
        out_specs=pl.BlockSpec((HI, C), lambda i: (0, 0)),
        out_shape=jax.ShapeDtypeStruct((HI, C), jnp.float32),
    )(dst2)


def _h0_body(x_ref, bv_ref, nt_ref, seed_ref, er_ref, wenc_ref, wts_ref,
             wtc_ref, benc_ref, out_ref):
    onehot = (bv_ref[...] == lax.broadcasted_iota(jnp.int32, (RB, B), 1)
              ).astype(jnp.float32)
    st = jnp.dot(onehot, seed_ref[...], preferred_element_type=jnp.float32)
    rel = st - nt_ref[...]
    k = lax.broadcasted_iota(jnp.int32, (1, C // 2), 1).astype(jnp.float32)
    freqs = jnp.exp(k * (-np.log(10000.0) / (C // 2)))
    ang = rel * freqs
    out_ref[...] = (
        jnp.dot(x_ref[...], wenc_ref[...], preferred_element_type=jnp.float32)
        + jnp.dot(jnp.sin(ang), wts_ref[...],
                  preferred_element_type=jnp.float32)
        + jnp.dot(jnp.cos(ang), wtc_ref[...],
                  preferred_element_type=jnp.float32)
        + benc_ref[...] + er_ref[...])


@jax.jit
def _h0(x_p, bv2, nt2, seed2, erows, W_enc, Wt_sin, Wt_cos, b_enc2):
    return pl.pallas_call(
        _h0_body,
        grid=(GRID,),
        in_specs=[
            pl.BlockSpec((RB, C), lambda i: (i, 0)),
            pl.BlockSpec((RB, 1), lambda i: (i, 0)),
            pl.BlockSpec((RB, 1), lambda i: (i, 0)),
            pl.BlockSpec((B, 1), lambda i: (0, 0)),
            pl.BlockSpec((RB, C), lambda i: (i, 0)),
            pl.BlockSpec((C, C), lambda i: (0, 0)),
            pl.BlockSpec((C // 2, C), lambda i: (0, 0)),
            pl.BlockSpec((C // 2, C), lambda i: (0, 0)),
            pl.BlockSpec((1, C), lambda i: (0, 0)),
        ],
        out_specs=pl.BlockSpec((RB, C), lambda i: (i, 0)),
        out_shape=jax.ShapeDtypeStruct((NPAD, C), jnp.float32),
    )(x_p, bv2, nt2, seed2, erows, W_enc, Wt_sin, Wt_cos, b_enc2)


def _layer_body(h_ref, p0_ref, p1_ref, deg_ref, ws_ref, wn_ref, bg_ref,
                out_ref):
    p = p0_ref[0] + p1_ref[0]
    deg = jnp.maximum(deg_ref[...], 1.0)
    mean = p / deg
    out_ref[...] = jnp.maximum(
        jnp.dot(h_ref[...], ws_ref[...], preferred_element_type=jnp.float32)
        + jnp.dot(mean, wn_ref[...], preferred_element_type=jnp.float32)
        + bg_ref[...], 0.0)


@jax.jit
def _layer(h, part, deg2, ws, wn, bg2):
    return pl.pallas_call(
        _layer_body,
        grid=(GRID,),
        in_specs=[
            pl.BlockSpec((RB, C), lambda i: (i, 0)),
            pl.BlockSpec((1, RB, C), lambda i: (0, i, 0)),
            pl.BlockSpec((1, RB, C), lambda i: (1, i, 0)),
            pl.BlockSpec((RB, 1), lambda i: (i, 0)),
            pl.BlockSpec((C, C), lambda i: (0, 0)),
            pl.BlockSpec((C, C), lambda i: (0, 0)),
            pl.BlockSpec((1, C), lambda i: (0, 0)),
        ],
        out_specs=pl.BlockSpec((RB, C), lambda i: (i, 0)),
        out_shape=jax.ShapeDtypeStruct((NPAD, C), jnp.float32),
    )(h, part, part, deg2, ws, wn, bg2)


def _head_body(h_ref, p0_ref, p1_ref, deg_ref, ws_ref, wn_ref, bg_ref, wh_ref,
               bh_ref, out_ref):
    p = p0_ref[0] + p1_ref[0]
    deg = jnp.maximum(deg_ref[...], 1.0)
    mean = p / deg
    h2 = jnp.maximum(
        jnp.dot(h_ref[...], ws_ref[...], preferred_element_type=jnp.float32)
        + jnp.dot(mean, wn_ref[...], preferred_element_type=jnp.float32)
        + bg_ref[...], 0.0)
    out_ref[...] = (jnp.dot(h2, wh_ref[...], preferred_element_type=jnp.float32)
                    + bh_ref[...])


@jax.jit
def _head(h, part2, deg2, ws, wn, bg2, W_head, b_head2):
    return pl.pallas_call(
        _head_body,
        grid=(1,),
        in_specs=[
            pl.BlockSpec((B, C), lambda i: (0, 0)),
            pl.BlockSpec((1, B, C), lambda i: (0, 0, 0)),
            pl.BlockSpec((1, B, C), lambda i: (1, 0, 0)),
            pl.BlockSpec((B, 1), lambda i: (0, 0)),
            pl.BlockSpec((C, C), lambda i: (0, 0)),
            pl.BlockSpec((C, C), lambda i: (0, 0)),
            pl.BlockSpec((1, C), lambda i: (0, 0)),
            pl.BlockSpec((C, 1), lambda i: (0, 0)),
            pl.BlockSpec((1, 1), lambda i: (0, 0)),
        ],
        out_specs=pl.BlockSpec((B, 1), lambda i: (0, 0)),
        out_shape=jax.ShapeDtypeStruct((B, 1), jnp.float32),
    )(h, part2, part2, deg2, ws, wn, bg2, W_head, b_head2)


def kernel(x, edge_index, n_id, node_time, seed_time, batch_vec,
           W_enc, b_enc, W_time, emb, W_self, W_neigh, b_gnn, W_head, b_head):
    pad = NPAD - N
    nid_p = jnp.concatenate([n_id, jnp.zeros((pad,), jnp.int32)])
    bv_p = jnp.concatenate([batch_vec, jnp.zeros((pad,), jnp.int32)])
    nt_p = jnp.concatenate([node_time, jnp.zeros((pad,), jnp.float32)])
    x_p = jnp.concatenate([x, jnp.zeros((pad, C), jnp.float32)])
    src = edge_index[0]
    dst = edge_index[1]
    sp = src
    dp = dst

    erows, fsrc, fdst, fcnt = _enc_gather(emb, nid_p, sp, dp)
    h0 = _h0(x_p, bv_p.reshape(NPAD, 1), nt_p.reshape(NPAD, 1),
             seed_time.reshape(B, 1), erows,
             W_enc, W_time[:C // 2], W_time[C // 2:], b_enc.reshape(1, C))
    part1 = _msg1(h0, sp, dp, jnp.zeros((RPT, C), jnp.float32))
    deg2 = _deg_histogram(dst.reshape(E, 1)).reshape(NPAD, 1)
    h1 = _layer(h0, part1, deg2, W_self[0], W_neigh[0], b_gnn[0].reshape(1, C))
    part2 = _msg2(h1, fsrc, fdst, fcnt,
                  jnp.zeros((A2R // NS, C), jnp.float32))
    return _head(h1, part2, deg2, W_self[1], W_neigh[1],
                 b_gnn[1].reshape(1, C), W_head, b_head.reshape(1, 1))
